# baseline (speedup 1.0000x reference)
import functools

import jax
import jax.numpy as jnp
from jax import lax
from jax.experimental import pallas as pl
from jax.experimental.pallas import tpu as pltpu
from jax.experimental.pallas import tpu_sc as plsc

N_NODES = 10000
N_EDGES = 160000
FEAT = 128
NODE_DIM = 128
EDGE_DIM = 128
HID_NODE = 128
HID_EDGE = 64
DEC_HID = 64
OUT_DIM = 128

_NC = 2
_NS = 16
_NW = _NC * _NS
_C = 128
_NCHUNK = N_EDGES // _C
_JT = 80
_N_PAD = 10240
_ROWS_PER_TILE = _N_PAD // _NS
_DUMMY_NODE = _N_PAD - 1

_NODE_TILE = 2000
_EDGE_TILE = 2000


def _silu(x):
    return x * (1.0 / (1.0 + jnp.exp(-x)))


def _ln(x, g, b):
    m = jnp.mean(x, axis=-1, keepdims=True)
    d = x - m
    v = jnp.mean(d * d, axis=-1, keepdims=True)
    return d * lax.rsqrt(v + 1e-5) * g + b


def _dot(a, b):
    return jnp.dot(a, b, preferred_element_type=jnp.float32)


def _tiled(rows, cols):
    return pl.BlockSpec((rows, cols), lambda i: (i, 0))


def _full(shape):
    return pl.BlockSpec(shape, lambda i: (0,) * len(shape))


def _enc_node_body(x, w1, b1, w2, b2, g, beta, wsd, h_o, hsd_o):
    h1 = _silu(_dot(x[...], w1[...]) + b1[...])
    h = _ln(_dot(h1, w2[...]) + b2[...], g[...], beta[...])
    h_o[...] = h
    hsd_o[...] = _dot(h, wsd[...])


def _enc_node_call(x, p, wsd):
    grid = N_NODES // _NODE_TILE
    return pl.pallas_call(
        _enc_node_body,
        grid=(grid,),
        in_specs=[
            _tiled(_NODE_TILE, FEAT),
            _full((FEAT, HID_NODE)), _full((1, HID_NODE)),
            _full((HID_NODE, NODE_DIM)), _full((1, NODE_DIM)),
            _full((1, NODE_DIM)), _full((1, NODE_DIM)),
            _full((NODE_DIM, 2 * HID_EDGE)),
        ],
        out_specs=(
            _tiled(_NODE_TILE, NODE_DIM),
            _tiled(_NODE_TILE, 2 * HID_EDGE),
        ),
        out_shape=(
            jax.ShapeDtypeStruct((N_NODES, NODE_DIM), jnp.float32),
            jax.ShapeDtypeStruct((N_NODES, 2 * HID_EDGE), jnp.float32),
        ),
    )(x, p["w1"], p["b1"].reshape(1, -1), p["w2"], p["b2"].reshape(1, -1),
      p["g"].reshape(1, -1), p["beta"].reshape(1, -1), wsd)


def _enc_edge_body(x, w1, b1, w2, b2, g, beta, e_o):
    h1 = _silu(_dot(x[...], w1[...]) + b1[...])
    e_o[...] = _ln(_dot(h1, w2[...]) + b2[...], g[...], beta[...])


def _enc_edge_call(x, p):
    grid = N_EDGES // _EDGE_TILE
    return pl.pallas_call(
        _enc_edge_body,
        grid=(grid,),
        in_specs=[
            _tiled(_EDGE_TILE, 4),
            _full((4, HID_EDGE)), _full((1, HID_EDGE)),
            _full((HID_EDGE, EDGE_DIM)), _full((1, EDGE_DIM)),
            _full((1, EDGE_DIM)), _full((1, EDGE_DIM)),
        ],
        out_specs=_tiled(_EDGE_TILE, EDGE_DIM),
        out_shape=jax.ShapeDtypeStruct((N_EDGES, EDGE_DIM), jnp.float32),
    )(x, p["w1"], p["b1"].reshape(1, -1), p["w2"], p["b2"].reshape(1, -1),
      p["g"].reshape(1, -1), p["beta"].reshape(1, -1))


def _edge_upd_body(e, gth, we, b1, w2, b2, g, beta, e_o):
    t = _dot(e[...], we[...]) + gth[0] + b1[...]
    upd = _ln(_dot(_silu(t), w2[...]) + b2[...], g[...], beta[...])
    e_o[...] = e[...] + upd


def _edge_upd_call(e, gth, m, we, p):
    grid = N_EDGES // _EDGE_TILE
    return pl.pallas_call(
        _edge_upd_body,
        grid=(grid,),
        in_specs=[
            _tiled(_EDGE_TILE, EDGE_DIM),
            pl.BlockSpec((1, _EDGE_TILE, HID_EDGE),
                         lambda i, m=m: (m, i, 0)),
            _full((EDGE_DIM, HID_EDGE)), _full((1, HID_EDGE)),
            _full((HID_EDGE, EDGE_DIM)), _full((1, EDGE_DIM)),
            _full((1, EDGE_DIM)), _full((1, EDGE_DIM)),
        ],
        out_specs=_tiled(_EDGE_TILE, EDGE_DIM),
        out_shape=jax.ShapeDtypeStruct((N_EDGES, EDGE_DIM), jnp.float32),
    )(e, gth, we, p["b1"].reshape(1, -1), p["w2"], p["b2"].reshape(1, -1),
      p["g"].reshape(1, -1), p["beta"].reshape(1, -1))


def _node_upd_proj_body(h, p2, wh, wa, b1, w2, b2, g, beta, wsd,
                        h_o, hsd_o):
    agg = p2[0]
    t = _dot(h[...], wh[...]) + _dot(agg, wa[...]) + b1[...]
    upd = _ln(_dot(_silu(t), w2[...]) + b2[...], g[...], beta[...])
    hn = h[...] + upd
    h_o[...] = hn
    hsd_o[...] = _dot(hn, wsd[...])


def _node_upd_last_body(h, p2, wh, wa, b1, w2, b2, g, beta, h_o):
    agg = p2[0]
    t = _dot(h[...], wh[...]) + _dot(agg, wa[...]) + b1[...]
    upd = _ln(_dot(_silu(t), w2[...]) + b2[...], g[...], beta[...])
    h_o[...] = h[...] + upd


def _node_upd_call(h, parts, m, p, wsd):
    grid = N_NODES // _NODE_TILE
    wh = p["w1"][:NODE_DIM]
    wa = p["w1"][NODE_DIM:]
    common_in = [
        _tiled(_NODE_TILE, NODE_DIM),
        pl.BlockSpec((1, _NODE_TILE, EDGE_DIM), lambda i, m=m: (m, i, 0)),
        _full((NODE_DIM, HID_NODE)), _full((EDGE_DIM, HID_NODE)),
        _full((1, HID_NODE)),
        _full((HID_NODE, NODE_DIM)), _full((1, NODE_DIM)),
        _full((1, NODE_DIM)), _full((1, NODE_DIM)),
    ]
    args = [h, parts, wh, wa, p["b1"].reshape(1, -1), p["w2"],
            p["b2"].reshape(1, -1), p["g"].reshape(1, -1),
            p["beta"].reshape(1, -1)]
    if wsd is None:
        return pl.pallas_call(
            _node_upd_last_body,
            grid=(grid,),
            in_specs=common_in,
            out_specs=_tiled(_NODE_TILE, NODE_DIM),
            out_shape=jax.ShapeDtypeStruct((N_NODES, NODE_DIM), jnp.float32),
        )(*args)
    return pl.pallas_call(
        _node_upd_proj_body,
        grid=(grid,),
        in_specs=common_in + [_full((NODE_DIM, 2 * HID_EDGE))],
        out_specs=(
            _tiled(_NODE_TILE, NODE_DIM),
            _tiled(_NODE_TILE, 2 * HID_EDGE),
        ),
        out_shape=(
            jax.ShapeDtypeStruct((N_NODES, NODE_DIM), jnp.float32),
            jax.ShapeDtypeStruct((N_NODES, 2 * HID_EDGE), jnp.float32),
        ),
    )(*(args + [wsd]))


def _decode_body(h0, h1, w1a, b1a, w2a, b2a, w1b, b1b, w2b, b2b, o):
    ya = _dot(_silu(_dot(h0[...], w1a[...]) + b1a[...]), w2a[...]) + b2a[...]
    yb = _dot(_silu(_dot(h1[...], w1b[...]) + b1b[...]), w2b[...]) + b2b[...]
    o[...] = ya + yb


def _decode_call(h0, h1, pa, pb, sp):
    grid = N_NODES // _NODE_TILE
    w2a = pa["w2"] * sp[0]
    b2a = pa["b2"].reshape(1, -1) * sp[0]
    w2b = pb["w2"] * sp[1]
    b2b = pb["b2"].reshape(1, -1) * sp[1]
    return pl.pallas_call(
        _decode_body,
        grid=(grid,),
        in_specs=[
            _tiled(_NODE_TILE, NODE_DIM), _tiled(_NODE_TILE, NODE_DIM),
            _full((NODE_DIM, DEC_HID)), _full((1, DEC_HID)),
            _full((DEC_HID, OUT_DIM)), _full((1, OUT_DIM)),
            _full((NODE_DIM, DEC_HID)), _full((1, DEC_HID)),
            _full((DEC_HID, OUT_DIM)), _full((1, OUT_DIM)),
        ],
        out_specs=_tiled(_NODE_TILE, OUT_DIM),
        out_shape=jax.ShapeDtypeStruct((N_NODES, OUT_DIM), jnp.float32),
    )(h0, h1, pa["w1"], pa["b1"].reshape(1, -1), w2a, b2a,
      pb["w1"], pb["b1"].reshape(1, -1), w2b, b2b)


@functools.lru_cache(maxsize=None)
def _sc_kernels():
    mesh = plsc.VectorSubcoreMesh(core_axis_name="c", subcore_axis_name="s",
                                  num_cores=_NC, num_subcores=_NS)

    @functools.partial(
        pl.kernel,
        mesh=mesh,
        out_type=jax.ShapeDtypeStruct((_NC, N_EDGES, HID_EDGE), jnp.float32),
        scratch_types=[
            pltpu.VMEM((_C,), jnp.int32),
            pltpu.VMEM((_C,), jnp.int32),
            pltpu.VMEM((_C,), jnp.int32),
            pltpu.VMEM((_C,), jnp.int32),
            pltpu.VMEM((_C, 2 * HID_EDGE), jnp.float32),
            pltpu.VMEM((_C, 2 * HID_EDGE), jnp.float32),
            pltpu.VMEM((_C, 2 * HID_EDGE), jnp.float32),
            pltpu.VMEM((_C, 2 * HID_EDGE), jnp.float32),
            pltpu.VMEM((_C, HID_EDGE), jnp.float32),
            pltpu.VMEM((_C, HID_EDGE), jnp.float32),
        ] + [pltpu.SemaphoreType.DMA] * 10,
    )
    def gather_add(hsd0_hbm, hsd1_hbm, src2_hbm, dst2_hbm, g_hbm,
                   idx_s0, idx_d0, idx_s1, idx_d1,
                   rs0, rs1, rd0, rd1, gb0, gb1,
                   si0, sj0, si1, sj1, ss0, ss1, sd0, sd1, sw0, sw1):
        cid = lax.axis_index("c")
        sid = lax.axis_index("s")
        nv = jnp.where(sid < _NCHUNK % _NS, _NCHUNK // _NS + 1,
                       _NCHUNK // _NS)
        slots = ((idx_s0, idx_d0, rs0, rd0, gb0, si0, sj0, ss0, sd0, sw0),
                 (idx_s1, idx_d1, rs1, rd1, gb1, si1, sj1, ss1, sd1, sw1))

        def idx_start(j, sl):
            base = (sid + _NS * j) * _C
            pltpu.async_copy(src2_hbm.at[pl.ds(base, _C)], sl[0], sl[5])
            pltpu.async_copy(dst2_hbm.at[pl.ds(base, _C)], sl[1], sl[6])

        def idx_wait(j, sl):
            base = (sid + _NS * j) * _C
            pltpu.make_async_copy(
                src2_hbm.at[pl.ds(base, _C)], sl[0], sl[5]).wait()
            pltpu.make_async_copy(
                dst2_hbm.at[pl.ds(base, _C)], sl[1], sl[6]).wait()

        def gath_start(sl):
            @pl.when(cid == 0)
            def _():
                pltpu.async_copy(hsd0_hbm.at[sl[0]], sl[2], sl[7])
                pltpu.async_copy(hsd0_hbm.at[sl[1]], sl[3], sl[8])

            @pl.when(cid == 1)
            def _():
                pltpu.async_copy(hsd1_hbm.at[sl[0]], sl[2], sl[7])
                pltpu.async_copy(hsd1_hbm.at[sl[1]], sl[3], sl[8])

        def gath_wait(sl):
            pltpu.make_async_copy(hsd0_hbm.at[sl[0]], sl[2], sl[7]).wait()
            pltpu.make_async_copy(hsd0_hbm.at[sl[1]], sl[3], sl[8]).wait()

        def wr_start(j, sl):
            base = (sid + _NS * j) * _C
            pltpu.async_copy(sl[4], g_hbm.at[cid, pl.ds(base, _C)], sl[9])

        def wr_wait(j, sl):
            base = (sid + _NS * j) * _C
            pltpu.make_async_copy(
                sl[4], g_hbm.at[cid, pl.ds(base, _C)], sl[9]).wait()

        for b in range(2):
            idx_start(b, slots[b])
        for b in range(2):
            idx_wait(b, slots[b])
            gath_start(slots[b])

        def body(jj, carry):
            for b in range(2):
                sl = slots[b]
                j = 2 * jj + b

                @pl.when(j < nv)
                def _():
                    gath_wait(sl)

                    @pl.when(j + 2 < nv)
                    def _():
                        idx_start(j + 2, sl)

                    @pl.when(jj > 0)
                    def _():
                        wr_wait(j - 2, sl)

                    def addrow(r, cr):
                        for u in range(HID_EDGE // 16):
                            lo = pl.ds(u * 16, 16)
                            hi = pl.ds(HID_EDGE + u * 16, 16)
                            sl[4][r, lo] = sl[2][r, lo] + sl[3][r, hi]
                        return cr

                    lax.fori_loop(0, _C, addrow, 0)
                    wr_start(j, sl)

                    @pl.when(j + 2 < nv)
                    def _():
                        idx_wait(j + 2, sl)
                        gath_start(sl)

                @pl.when(jnp.logical_and(j >= nv, jj > 0))
                def _():
                    wr_wait(j - 2, sl)

            return carry

        lax.fori_loop(0, _JT // 2, body, 0)

        @pl.when(nv > _JT - 2)
        def _():
            wr_wait(_JT - 2, slots[0])

        @pl.when(nv > _JT - 1)
        def _():
            wr_wait(_JT - 1, slots[1])

    @functools.partial(
        pl.kernel,
        mesh=mesh,
        out_type=jax.ShapeDtypeStruct((_NC, _N_PAD, EDGE_DIM), jnp.float32),
        scratch_types=[
            pltpu.VMEM((_C,), jnp.int32),
            pltpu.VMEM((_C,), jnp.int32),
            pltpu.VMEM((_C, EDGE_DIM), jnp.float32),
            pltpu.VMEM((_C, EDGE_DIM), jnp.float32),
            pltpu.VMEM_SHARED((_N_PAD, EDGE_DIM), jnp.float32),
            pltpu.SemaphoreType.DMA,
            pltpu.SemaphoreType.DMA,
            pltpu.SemaphoreType.DMA,
            pltpu.SemaphoreType.DMA,
        ],
    )
    def segsum(e0_hbm, e1_hbm, dst2_hbm, p_hbm, idx0, idx1, eb0, eb1, acc,
               si0, si1, se0, se1):
        cid = lax.axis_index("c")
        sid = lax.axis_index("s")
        nv = jnp.where(sid < _NCHUNK % _NS, _NCHUNK // _NS + 1,
                       _NCHUNK // _NS)
        slots = ((idx0, eb0, si0, se0), (idx1, eb1, si1, se1))

        def ld_start(j, sl):
            base = (sid + _NS * j) * _C
            pltpu.async_copy(dst2_hbm.at[pl.ds(base, _C)], sl[0], sl[2])

            @pl.when(cid == 0)
            def _():
                pltpu.async_copy(e0_hbm.at[pl.ds(base, _C)], sl[1], sl[3])

            @pl.when(cid == 1)
            def _():
                pltpu.async_copy(e1_hbm.at[pl.ds(base, _C)], sl[1], sl[3])

        def ld_wait(j, sl):
            base = (sid + _NS * j) * _C
            pltpu.make_async_copy(
                dst2_hbm.at[pl.ds(base, _C)], sl[0], sl[2]).wait()
            pltpu.make_async_copy(
                e0_hbm.at[pl.ds(base, _C)], sl[1], sl[3]).wait()

        zv = jnp.zeros((16,), jnp.float32)

        def zrow(r, cr):
            for u in range(EDGE_DIM // 16):
                eb0[r, pl.ds(u * 16, 16)] = zv
            return cr

        lax.fori_loop(0, 128, zrow, 0)
        for t in range(_ROWS_PER_TILE // 128):
            pltpu.sync_copy(eb0,
                            acc.at[pl.ds(sid * _ROWS_PER_TILE + t * 128, 128)])
        plsc.subcore_barrier()

        for b in range(2):
            ld_start(b, slots[b])

        def body(jj, carry):
            for b in range(2):
                sl = slots[b]
                j = 2 * jj + b

                @pl.when(j < nv)
                def _():
                    ld_wait(j, sl)
                    pltpu.sync_copy(sl[1], acc.at[sl[0]], add=True)

                    @pl.when(j + 2 < nv)
                    def _():
                        ld_start(j + 2, sl)

            return carry

        lax.fori_loop(0, _JT // 2, body, 0)
        plsc.subcore_barrier()
        pltpu.sync_copy(
            acc.at[pl.ds(sid * _ROWS_PER_TILE, _ROWS_PER_TILE)],
            p_hbm.at[cid, pl.ds(sid * _ROWS_PER_TILE, _ROWS_PER_TILE)])

    return gather_add, segsum


def _gather_add2(hsd0, hsd1, src, dst):
    return _sc_kernels()[0](hsd0, hsd1, src, dst)


def _segsum2(e0, e1, dst):
    return _sc_kernels()[1](e0, e1, dst)


def _split_edge_w1(w1):
    return w1[:NODE_DIM], w1[NODE_DIM:2 * NODE_DIM], w1[2 * NODE_DIM:]


def kernel(features, edge_index, edge_attr, params):
    src = edge_index[0]
    dst = edge_index[1]
    feats0 = features[0]
    sp = params["step_params"]
    models = params["models"]


    nm = len(models)
    blks = [m["blocks"] for m in models]
    we = [[_split_edge_w1(blks[i][b]["edge"]["w1"])[2] for b in range(2)]
          for i in range(nm)]
    wsd = [[jnp.concatenate(_split_edge_w1(blks[i][b]["edge"]["w1"])[:2],
                            axis=1) for b in range(2)]
           for i in range(nm)]

    h, hsd, e = [None] * nm, [None] * nm, [None] * nm
    for i in range(nm):
        h[i], hsd[i] = _enc_node_call(feats0[i], models[i]["enc_node"],
                                      wsd[i][0])
        e[i] = _enc_edge_call(edge_attr, models[i]["enc_edge"])

    for b in range(2):
        g = _gather_add2(hsd[0], hsd[1], src, dst)
        for i in range(nm):
            e[i] = _edge_upd_call(e[i], g, i, we[i][b], blks[i][b]["edge"])
        p = _segsum2(e[0], e[1], dst)
        for i in range(nm):
            if b == 0:
                h[i], hsd[i] = _node_upd_call(h[i], p, i, blks[i][b]["node"],
                                              wsd[i][1])
            else:
                h[i] = _node_upd_call(h[i], p, i, blks[i][b]["node"], None)

    out = _decode_call(h[0], h[1], models[0]["dec"], models[1]["dec"], sp)
    return out[None]


# speedup vs baseline: 1.2161x; 1.2161x over previous
import functools

import jax
import jax.numpy as jnp
from jax import lax
from jax.experimental import pallas as pl
from jax.experimental.pallas import tpu as pltpu
from jax.experimental.pallas import tpu_sc as plsc

N_NODES = 10000
N_EDGES = 160000
FEAT = 128
NODE_DIM = 128
EDGE_DIM = 128
HID_NODE = 128
HID_EDGE = 64
DEC_HID = 64
OUT_DIM = 128

_NC = 2
_NS = 16
_NW = _NC * _NS
_C = 128
_JT = 40
_NCHUNK = N_EDGES // _C
_N_PAD = 10240
_ROWS_PER_TILE = _N_PAD // _NS
_DUMMY_NODE = _N_PAD - 1

_NODE_TILE = 2000
_EDGE_TILE = 2000


def _silu(x):
    return x * (1.0 / (1.0 + jnp.exp(-x)))


def _ln(x, g, b):
    m = jnp.mean(x, axis=-1, keepdims=True)
    d = x - m
    v = jnp.mean(d * d, axis=-1, keepdims=True)
    return d * lax.rsqrt(v + 1e-5) * g + b


def _dot(a, b):
    return jnp.dot(a, b, preferred_element_type=jnp.float32)


def _tiled(rows, cols):
    return pl.BlockSpec((rows, cols), lambda i: (i, 0))


def _full(shape):
    return pl.BlockSpec(shape, lambda i: (0,) * len(shape))


def _enc_node_body(x, w1, b1, w2, b2, g, beta, wsd, h_o, hsd_o):
    h1 = _silu(_dot(x[...], w1[...]) + b1[...])
    h = _ln(_dot(h1, w2[...]) + b2[...], g[...], beta[...])
    h_o[...] = h
    hsd_o[...] = _dot(h, wsd[...])


def _enc_node_call(x, p, wsd):
    grid = N_NODES // _NODE_TILE
    return pl.pallas_call(
        _enc_node_body,
        grid=(grid,),
        in_specs=[
            _tiled(_NODE_TILE, FEAT),
            _full((FEAT, HID_NODE)), _full((1, HID_NODE)),
            _full((HID_NODE, NODE_DIM)), _full((1, NODE_DIM)),
            _full((1, NODE_DIM)), _full((1, NODE_DIM)),
            _full((NODE_DIM, 2 * HID_EDGE)),
        ],
        out_specs=(
            _tiled(_NODE_TILE, NODE_DIM),
            _tiled(_NODE_TILE, 2 * HID_EDGE),
        ),
        out_shape=(
            jax.ShapeDtypeStruct((N_NODES, NODE_DIM), jnp.float32),
            jax.ShapeDtypeStruct((N_NODES, 2 * HID_EDGE), jnp.float32),
        ),
    )(x, p["w1"], p["b1"].reshape(1, -1), p["w2"], p["b2"].reshape(1, -1),
      p["g"].reshape(1, -1), p["beta"].reshape(1, -1), wsd)


def _enc_edge_body(x, w1, b1, w2, b2, g, beta, e_o):
    h1 = _silu(_dot(x[...], w1[...]) + b1[...])
    e_o[...] = _ln(_dot(h1, w2[...]) + b2[...], g[...], beta[...])


def _enc_edge_call(x, p):
    grid = N_EDGES // _EDGE_TILE
    return pl.pallas_call(
        _enc_edge_body,
        grid=(grid,),
        in_specs=[
            _tiled(_EDGE_TILE, 4),
            _full((4, HID_EDGE)), _full((1, HID_EDGE)),
            _full((HID_EDGE, EDGE_DIM)), _full((1, EDGE_DIM)),
            _full((1, EDGE_DIM)), _full((1, EDGE_DIM)),
        ],
        out_specs=_tiled(_EDGE_TILE, EDGE_DIM),
        out_shape=jax.ShapeDtypeStruct((N_EDGES, EDGE_DIM), jnp.float32),
    )(x, p["w1"], p["b1"].reshape(1, -1), p["w2"], p["b2"].reshape(1, -1),
      p["g"].reshape(1, -1), p["beta"].reshape(1, -1))


def _edge_enc_upd_body(x, gth, w1e, b1e, w2e, b2e, ge, betae,
                       we, b1, w2, b2, g, beta, e_o):
    enc = _ln(_dot(_silu(_dot(x[...], w1e[...]) + b1e[...]), w2e[...])
              + b2e[...], ge[...], betae[...])
    t = _dot(enc, we[...]) + gth[...] + b1[...]
    upd = _ln(_dot(_silu(t), w2[...]) + b2[...], g[...], beta[...])
    e_o[...] = enc + upd


def _edge_enc_upd_call(x, gth, pe, we, p):
    grid = N_EDGES // _EDGE_TILE
    return pl.pallas_call(
        _edge_enc_upd_body,
        grid=(grid,),
        in_specs=[
            _tiled(_EDGE_TILE, 4),
            _tiled(_EDGE_TILE, HID_EDGE),
            _full((4, HID_EDGE)), _full((1, HID_EDGE)),
            _full((HID_EDGE, EDGE_DIM)), _full((1, EDGE_DIM)),
            _full((1, EDGE_DIM)), _full((1, EDGE_DIM)),
            _full((EDGE_DIM, HID_EDGE)), _full((1, HID_EDGE)),
            _full((HID_EDGE, EDGE_DIM)), _full((1, EDGE_DIM)),
            _full((1, EDGE_DIM)), _full((1, EDGE_DIM)),
        ],
        out_specs=_tiled(_EDGE_TILE, EDGE_DIM),
        out_shape=jax.ShapeDtypeStruct((N_EDGES, EDGE_DIM), jnp.float32),
    )(x, gth,
      pe["w1"], pe["b1"].reshape(1, -1), pe["w2"], pe["b2"].reshape(1, -1),
      pe["g"].reshape(1, -1), pe["beta"].reshape(1, -1),
      we, p["b1"].reshape(1, -1), p["w2"], p["b2"].reshape(1, -1),
      p["g"].reshape(1, -1), p["beta"].reshape(1, -1))


def _edge_upd_body(e, gth, we, b1, w2, b2, g, beta, e_o):
    t = _dot(e[...], we[...]) + gth[...] + b1[...]
    upd = _ln(_dot(_silu(t), w2[...]) + b2[...], g[...], beta[...])
    e_o[...] = e[...] + upd


def _edge_upd_call(e, gth, we, p):
    grid = N_EDGES // _EDGE_TILE
    return pl.pallas_call(
        _edge_upd_body,
        grid=(grid,),
        in_specs=[
            _tiled(_EDGE_TILE, EDGE_DIM),
            _tiled(_EDGE_TILE, HID_EDGE),
            _full((EDGE_DIM, HID_EDGE)), _full((1, HID_EDGE)),
            _full((HID_EDGE, EDGE_DIM)), _full((1, EDGE_DIM)),
            _full((1, EDGE_DIM)), _full((1, EDGE_DIM)),
        ],
        out_specs=_tiled(_EDGE_TILE, EDGE_DIM),
        out_shape=jax.ShapeDtypeStruct((N_EDGES, EDGE_DIM), jnp.float32),
    )(e, gth, we, p["b1"].reshape(1, -1), p["w2"], p["b2"].reshape(1, -1),
      p["g"].reshape(1, -1), p["beta"].reshape(1, -1))


def _node_upd_proj_body(h, p2, wh, wa, b1, w2, b2, g, beta, wsd,
                        h_o, hsd_o):
    agg = p2[0] + p2[1]
    t = _dot(h[...], wh[...]) + _dot(agg, wa[...]) + b1[...]
    upd = _ln(_dot(_silu(t), w2[...]) + b2[...], g[...], beta[...])
    hn = h[...] + upd
    h_o[...] = hn
    hsd_o[...] = _dot(hn, wsd[...])


def _node_upd_last_body(h, p2, wh, wa, b1, w2, b2, g, beta, h_o):
    agg = p2[0] + p2[1]
    t = _dot(h[...], wh[...]) + _dot(agg, wa[...]) + b1[...]
    upd = _ln(_dot(_silu(t), w2[...]) + b2[...], g[...], beta[...])
    h_o[...] = h[...] + upd


def _node_upd_call(h, parts, p, wsd):
    grid = N_NODES // _NODE_TILE
    wh = p["w1"][:NODE_DIM]
    wa = p["w1"][NODE_DIM:]
    common_in = [
        _tiled(_NODE_TILE, NODE_DIM),
        pl.BlockSpec((2, _NODE_TILE, EDGE_DIM), lambda i: (0, i, 0)),
        _full((NODE_DIM, HID_NODE)), _full((EDGE_DIM, HID_NODE)),
        _full((1, HID_NODE)),
        _full((HID_NODE, NODE_DIM)), _full((1, NODE_DIM)),
        _full((1, NODE_DIM)), _full((1, NODE_DIM)),
    ]
    args = [h, parts, wh, wa, p["b1"].reshape(1, -1), p["w2"],
            p["b2"].reshape(1, -1), p["g"].reshape(1, -1),
            p["beta"].reshape(1, -1)]
    if wsd is None:
        return pl.pallas_call(
            _node_upd_last_body,
            grid=(grid,),
            in_specs=common_in,
            out_specs=_tiled(_NODE_TILE, NODE_DIM),
            out_shape=jax.ShapeDtypeStruct((N_NODES, NODE_DIM), jnp.float32),
        )(*args)
    return pl.pallas_call(
        _node_upd_proj_body,
        grid=(grid,),
        in_specs=common_in + [_full((NODE_DIM, 2 * HID_EDGE))],
        out_specs=(
            _tiled(_NODE_TILE, NODE_DIM),
            _tiled(_NODE_TILE, 2 * HID_EDGE),
        ),
        out_shape=(
            jax.ShapeDtypeStruct((N_NODES, NODE_DIM), jnp.float32),
            jax.ShapeDtypeStruct((N_NODES, 2 * HID_EDGE), jnp.float32),
        ),
    )(*(args + [wsd]))


def _decode_body(h0, h1, w1a, b1a, w2a, b2a, w1b, b1b, w2b, b2b, o):
    ya = _dot(_silu(_dot(h0[...], w1a[...]) + b1a[...]), w2a[...]) + b2a[...]
    yb = _dot(_silu(_dot(h1[...], w1b[...]) + b1b[...]), w2b[...]) + b2b[...]
    o[...] = ya + yb


def _decode_call(h0, h1, pa, pb, sp):
    grid = N_NODES // _NODE_TILE
    w2a = pa["w2"] * sp[0]
    b2a = pa["b2"].reshape(1, -1) * sp[0]
    w2b = pb["w2"] * sp[1]
    b2b = pb["b2"].reshape(1, -1) * sp[1]
    return pl.pallas_call(
        _decode_body,
        grid=(grid,),
        in_specs=[
            _tiled(_NODE_TILE, NODE_DIM), _tiled(_NODE_TILE, NODE_DIM),
            _full((NODE_DIM, DEC_HID)), _full((1, DEC_HID)),
            _full((DEC_HID, OUT_DIM)), _full((1, OUT_DIM)),
            _full((NODE_DIM, DEC_HID)), _full((1, DEC_HID)),
            _full((DEC_HID, OUT_DIM)), _full((1, OUT_DIM)),
        ],
        out_specs=_tiled(_NODE_TILE, OUT_DIM),
        out_shape=jax.ShapeDtypeStruct((N_NODES, OUT_DIM), jnp.float32),
    )(h0, h1, pa["w1"], pa["b1"].reshape(1, -1), w2a, b2a,
      pb["w1"], pb["b1"].reshape(1, -1), w2b, b2b)


@functools.lru_cache(maxsize=None)
def _sc_kernels():
    mesh = plsc.VectorSubcoreMesh(core_axis_name="c", subcore_axis_name="s",
                                  num_cores=_NC, num_subcores=_NS)

    @functools.partial(
        pl.kernel,
        mesh=mesh,
        out_type=jax.ShapeDtypeStruct((N_EDGES, HID_EDGE), jnp.float32),
        scratch_types=[
            pltpu.VMEM((_C,), jnp.int32),
            pltpu.VMEM((_C,), jnp.int32),
            pltpu.VMEM((_C,), jnp.int32),
            pltpu.VMEM((_C,), jnp.int32),
            pltpu.VMEM((_C, 2 * HID_EDGE), jnp.float32),
            pltpu.VMEM((_C, 2 * HID_EDGE), jnp.float32),
            pltpu.VMEM((_C, 2 * HID_EDGE), jnp.float32),
            pltpu.VMEM((_C, 2 * HID_EDGE), jnp.float32),
            pltpu.VMEM((_C, HID_EDGE), jnp.float32),
            pltpu.VMEM((_C, HID_EDGE), jnp.float32),
        ] + [pltpu.SemaphoreType.DMA] * 10,
    )
    def gather_add(hsd_hbm, src2_hbm, dst2_hbm, g_hbm,
                   idx_s0, idx_d0, idx_s1, idx_d1,
                   rs0, rs1, rd0, rd1, gb0, gb1,
                   si0, sj0, si1, sj1, ss0, ss1, sd0, sd1, sw0, sw1):
        wid = lax.axis_index("s") * _NC + lax.axis_index("c")
        nv = jnp.where(wid < _NCHUNK - (_JT - 1) * _NW, _JT, _JT - 1)
        slots = ((idx_s0, idx_d0, rs0, rd0, gb0, si0, sj0, ss0, sd0, sw0),
                 (idx_s1, idx_d1, rs1, rd1, gb1, si1, sj1, ss1, sd1, sw1))

        def idx_start(j, sl):
            base = (wid + _NW * j) * _C
            pltpu.async_copy(src2_hbm.at[pl.ds(base, _C)], sl[0], sl[5])
            pltpu.async_copy(dst2_hbm.at[pl.ds(base, _C)], sl[1], sl[6])

        def idx_wait(j, sl):
            base = (wid + _NW * j) * _C
            pltpu.make_async_copy(
                src2_hbm.at[pl.ds(base, _C)], sl[0], sl[5]).wait()
            pltpu.make_async_copy(
                dst2_hbm.at[pl.ds(base, _C)], sl[1], sl[6]).wait()

        def gath_start(sl):
            pltpu.async_copy(hsd_hbm.at[sl[0]], sl[2], sl[7])
            pltpu.async_copy(hsd_hbm.at[sl[1]], sl[3], sl[8])

        def gath_wait(sl):
            pltpu.make_async_copy(hsd_hbm.at[sl[0]], sl[2], sl[7]).wait()
            pltpu.make_async_copy(hsd_hbm.at[sl[1]], sl[3], sl[8]).wait()

        def wr_start(j, sl):
            base = (wid + _NW * j) * _C
            pltpu.async_copy(sl[4], g_hbm.at[pl.ds(base, _C)], sl[9])

        def wr_wait(j, sl):
            base = (wid + _NW * j) * _C
            pltpu.make_async_copy(
                sl[4], g_hbm.at[pl.ds(base, _C)], sl[9]).wait()

        for b in range(2):
            idx_start(b, slots[b])
        for b in range(2):
            idx_wait(b, slots[b])
            gath_start(slots[b])

        def body(jj, carry):
            for b in range(2):
                sl = slots[b]
                j = 2 * jj + b

                @pl.when(j < nv)
                def _():
                    gath_wait(sl)

                    @pl.when(j + 2 < nv)
                    def _():
                        idx_start(j + 2, sl)

                    @pl.when(jj > 0)
                    def _():
                        wr_wait(j - 2, sl)

                    def addrow(r, cr):
                        for u in range(HID_EDGE // 16):
                            lo = pl.ds(u * 16, 16)
                            hi = pl.ds(HID_EDGE + u * 16, 16)
                            sl[4][r, lo] = sl[2][r, lo] + sl[3][r, hi]
                        return cr

                    lax.fori_loop(0, _C, addrow, 0, unroll=4)
                    wr_start(j, sl)

                    @pl.when(j + 2 < nv)
                    def _():
                        idx_wait(j + 2, sl)
                        gath_start(sl)

                @pl.when(jnp.logical_and(j >= nv, jj > 0))
                def _():
                    wr_wait(j - 2, sl)

            return carry

        lax.fori_loop(0, _JT // 2, body, 0)
        wr_wait(_JT - 2, slots[0])

        @pl.when(nv > _JT - 1)
        def _():
            wr_wait(_JT - 1, slots[1])

    @functools.partial(
        pl.kernel,
        mesh=mesh,
        out_type=jax.ShapeDtypeStruct((_NC, _N_PAD, EDGE_DIM), jnp.float32),
        scratch_types=[
            pltpu.VMEM((_C,), jnp.int32),
            pltpu.VMEM((_C,), jnp.int32),
            pltpu.VMEM((_C, EDGE_DIM), jnp.float32),
            pltpu.VMEM((_C, EDGE_DIM), jnp.float32),
            pltpu.VMEM_SHARED((_N_PAD, EDGE_DIM), jnp.float32),
            pltpu.SemaphoreType.DMA,
            pltpu.SemaphoreType.DMA,
            pltpu.SemaphoreType.DMA,
            pltpu.SemaphoreType.DMA,
        ],
    )
    def segsum(e_hbm, dst2_hbm, p_hbm, idx0, idx1, eb0, eb1, acc,
               si0, si1, se0, se1):
        cid = lax.axis_index("c")
        sid = lax.axis_index("s")
        wid = sid * _NC + cid
        nv = jnp.where(wid < _NCHUNK - (_JT - 1) * _NW, _JT, _JT - 1)
        slots = ((idx0, eb0, si0, se0), (idx1, eb1, si1, se1))

        def ld_start(j, sl):
            base = (wid + _NW * j) * _C
            pltpu.async_copy(dst2_hbm.at[pl.ds(base, _C)], sl[0], sl[2])
            pltpu.async_copy(e_hbm.at[pl.ds(base, _C)], sl[1], sl[3])

        def ld_wait(j, sl):
            base = (wid + _NW * j) * _C
            pltpu.make_async_copy(
                dst2_hbm.at[pl.ds(base, _C)], sl[0], sl[2]).wait()
            pltpu.make_async_copy(
                e_hbm.at[pl.ds(base, _C)], sl[1], sl[3]).wait()

        zv = jnp.zeros((16,), jnp.float32)

        def zrow(r, cr):
            for u in range(EDGE_DIM // 16):
                eb0[r, pl.ds(u * 16, 16)] = zv
            return cr

        lax.fori_loop(0, 128, zrow, 0)
        for t in range(_ROWS_PER_TILE // 128):
            pltpu.sync_copy(eb0,
                            acc.at[pl.ds(sid * _ROWS_PER_TILE + t * 128, 128)])
        plsc.subcore_barrier()

        for b in range(2):
            ld_start(b, slots[b])

        def body(jj, carry):
            for b in range(2):
                sl = slots[b]
                j = 2 * jj + b

                @pl.when(j < nv)
                def _():
                    ld_wait(j, sl)
                    pltpu.sync_copy(sl[1], acc.at[sl[0]], add=True)

                    @pl.when(j + 2 < nv)
                    def _():
                        ld_start(j + 2, sl)

            return carry

        lax.fori_loop(0, _JT // 2, body, 0)
        plsc.subcore_barrier()
        pltpu.sync_copy(
            acc.at[pl.ds(sid * _ROWS_PER_TILE, _ROWS_PER_TILE)],
            p_hbm.at[cid, pl.ds(sid * _ROWS_PER_TILE, _ROWS_PER_TILE)])

    return gather_add, segsum


def _gather_add(hsd, src, dst):
    return _sc_kernels()[0](hsd, src, dst)


def _segsum(e, dst):
    return _sc_kernels()[1](e, dst)


def _split_edge_w1(w1):
    return w1[:NODE_DIM], w1[NODE_DIM:2 * NODE_DIM], w1[2 * NODE_DIM:]


def kernel(features, edge_index, edge_attr, params):
    src = edge_index[0]
    dst = edge_index[1]
    feats0 = features[0]
    sp = params["step_params"]
    models = params["models"]


    h_final = []
    for i in range(len(models)):
        mp = models[i]
        blk0, blk1 = mp["blocks"]
        ws0, wd0, we0 = _split_edge_w1(blk0["edge"]["w1"])
        ws1, wd1, we1 = _split_edge_w1(blk1["edge"]["w1"])
        wsd0 = jnp.concatenate([ws0, wd0], axis=1)
        wsd1 = jnp.concatenate([ws1, wd1], axis=1)

        h, hsd = _enc_node_call(feats0[i], mp["enc_node"], wsd0)

        g = _gather_add(hsd, src, dst)
        e = _edge_enc_upd_call(edge_attr, g, mp["enc_edge"], we0,
                               blk0["edge"])
        parts = _segsum(e, dst)
        h, hsd = _node_upd_call(h, parts, blk0["node"], wsd1)

        g = _gather_add(hsd, src, dst)
        e = _edge_upd_call(e, g, we1, blk1["edge"])
        parts = _segsum(e, dst)
        h = _node_upd_call(h, parts, blk1["node"], None)

        h_final.append(h)

    out = _decode_call(h_final[0], h_final[1], models[0]["dec"],
                       models[1]["dec"], sp)
    return out[None]


# speedup vs baseline: 1.2191x; 1.0025x over previous
import functools

import jax
import jax.numpy as jnp
from jax import lax
from jax.experimental import pallas as pl
from jax.experimental.pallas import tpu as pltpu
from jax.experimental.pallas import tpu_sc as plsc

N_NODES = 10000
N_EDGES = 160000
FEAT = 128
NODE_DIM = 128
EDGE_DIM = 128
HID_NODE = 128
HID_EDGE = 64
DEC_HID = 64
OUT_DIM = 128

_NC = 2
_NS = 16
_NW = _NC * _NS
_C = 128
_JT = 40
_NCHUNK = N_EDGES // _C
_N_PAD = 10240
_ROWS_PER_TILE = _N_PAD // _NS
_DUMMY_NODE = _N_PAD - 1

_NODE_TILE = 2000
_EDGE_TILE = 2000


def _silu(x):
    return x * (1.0 / (1.0 + jnp.exp(-x)))


def _ln(x, g, b):
    m = jnp.mean(x, axis=-1, keepdims=True)
    d = x - m
    v = jnp.mean(d * d, axis=-1, keepdims=True)
    return d * lax.rsqrt(v + 1e-5) * g + b


def _dot(a, b):
    return jnp.dot(a, b, preferred_element_type=jnp.float32)


def _tiled(rows, cols):
    return pl.BlockSpec((rows, cols), lambda i: (i, 0))


def _full(shape):
    return pl.BlockSpec(shape, lambda i: (0,) * len(shape))


def _enc_node_body(x, w1, b1, w2, b2, g, beta, wsd, h_o, hsd_o):
    h1 = _silu(_dot(x[...], w1[...]) + b1[...])
    h = _ln(_dot(h1, w2[...]) + b2[...], g[...], beta[...])
    h_o[...] = h
    hsd_o[...] = _dot(h, wsd[...])


def _enc_node_call(x, p, wsd):
    grid = N_NODES // _NODE_TILE
    return pl.pallas_call(
        _enc_node_body,
        grid=(grid,),
        in_specs=[
            _tiled(_NODE_TILE, FEAT),
            _full((FEAT, HID_NODE)), _full((1, HID_NODE)),
            _full((HID_NODE, NODE_DIM)), _full((1, NODE_DIM)),
            _full((1, NODE_DIM)), _full((1, NODE_DIM)),
            _full((NODE_DIM, 2 * HID_EDGE)),
        ],
        out_specs=(
            _tiled(_NODE_TILE, NODE_DIM),
            _tiled(_NODE_TILE, 2 * HID_EDGE),
        ),
        out_shape=(
            jax.ShapeDtypeStruct((N_NODES, NODE_DIM), jnp.float32),
            jax.ShapeDtypeStruct((N_NODES, 2 * HID_EDGE), jnp.float32),
        ),
    )(x, p["w1"], p["b1"].reshape(1, -1), p["w2"], p["b2"].reshape(1, -1),
      p["g"].reshape(1, -1), p["beta"].reshape(1, -1), wsd)


def _enc_edge_body(x, w1, b1, w2, b2, g, beta, e_o):
    h1 = _silu(_dot(x[...], w1[...]) + b1[...])
    e_o[...] = _ln(_dot(h1, w2[...]) + b2[...], g[...], beta[...])


def _enc_edge_call(x, p):
    grid = N_EDGES // _EDGE_TILE
    return pl.pallas_call(
        _enc_edge_body,
        grid=(grid,),
        in_specs=[
            _tiled(_EDGE_TILE, 4),
            _full((4, HID_EDGE)), _full((1, HID_EDGE)),
            _full((HID_EDGE, EDGE_DIM)), _full((1, EDGE_DIM)),
            _full((1, EDGE_DIM)), _full((1, EDGE_DIM)),
        ],
        out_specs=_tiled(_EDGE_TILE, EDGE_DIM),
        out_shape=jax.ShapeDtypeStruct((N_EDGES, EDGE_DIM), jnp.float32),
    )(x, p["w1"], p["b1"].reshape(1, -1), p["w2"], p["b2"].reshape(1, -1),
      p["g"].reshape(1, -1), p["beta"].reshape(1, -1))


def _edge_enc_upd_body(x, gth, w1e, b1e, w2e, b2e, ge, betae,
                       we, b1, w2, b2, g, beta, e_o):
    enc = _ln(_dot(_silu(_dot(x[...], w1e[...]) + b1e[...]), w2e[...])
              + b2e[...], ge[...], betae[...])
    t = _dot(enc, we[...]) + gth[...] + b1[...]
    upd = _ln(_dot(_silu(t), w2[...]) + b2[...], g[...], beta[...])
    e_o[...] = enc + upd


def _edge_enc_upd_call(x, gth, pe, we, p):
    grid = N_EDGES // _EDGE_TILE
    return pl.pallas_call(
        _edge_enc_upd_body,
        grid=(grid,),
        in_specs=[
            _tiled(_EDGE_TILE, 4),
            _tiled(_EDGE_TILE, HID_EDGE),
            _full((4, HID_EDGE)), _full((1, HID_EDGE)),
            _full((HID_EDGE, EDGE_DIM)), _full((1, EDGE_DIM)),
            _full((1, EDGE_DIM)), _full((1, EDGE_DIM)),
            _full((EDGE_DIM, HID_EDGE)), _full((1, HID_EDGE)),
            _full((HID_EDGE, EDGE_DIM)), _full((1, EDGE_DIM)),
            _full((1, EDGE_DIM)), _full((1, EDGE_DIM)),
        ],
        out_specs=_tiled(_EDGE_TILE, EDGE_DIM),
        out_shape=jax.ShapeDtypeStruct((N_EDGES, EDGE_DIM), jnp.float32),
    )(x, gth,
      pe["w1"], pe["b1"].reshape(1, -1), pe["w2"], pe["b2"].reshape(1, -1),
      pe["g"].reshape(1, -1), pe["beta"].reshape(1, -1),
      we, p["b1"].reshape(1, -1), p["w2"], p["b2"].reshape(1, -1),
      p["g"].reshape(1, -1), p["beta"].reshape(1, -1))


def _edge_upd_body(e, gth, we, b1, w2, b2, g, beta, e_o):
    t = _dot(e[...], we[...]) + gth[...] + b1[...]
    upd = _ln(_dot(_silu(t), w2[...]) + b2[...], g[...], beta[...])
    e_o[...] = e[...] + upd


def _edge_upd_call(e, gth, we, p):
    grid = N_EDGES // _EDGE_TILE
    return pl.pallas_call(
        _edge_upd_body,
        grid=(grid,),
        in_specs=[
            _tiled(_EDGE_TILE, EDGE_DIM),
            _tiled(_EDGE_TILE, HID_EDGE),
            _full((EDGE_DIM, HID_EDGE)), _full((1, HID_EDGE)),
            _full((HID_EDGE, EDGE_DIM)), _full((1, EDGE_DIM)),
            _full((1, EDGE_DIM)), _full((1, EDGE_DIM)),
        ],
        out_specs=_tiled(_EDGE_TILE, EDGE_DIM),
        out_shape=jax.ShapeDtypeStruct((N_EDGES, EDGE_DIM), jnp.float32),
    )(e, gth, we, p["b1"].reshape(1, -1), p["w2"], p["b2"].reshape(1, -1),
      p["g"].reshape(1, -1), p["beta"].reshape(1, -1))


def _node_upd_proj_body(h, p2, wh, wa, b1, w2, b2, g, beta, wsd,
                        h_o, hsd_o):
    agg = p2[0] + p2[1]
    t = _dot(h[...], wh[...]) + _dot(agg, wa[...]) + b1[...]
    upd = _ln(_dot(_silu(t), w2[...]) + b2[...], g[...], beta[...])
    hn = h[...] + upd
    h_o[...] = hn
    hsd_o[...] = _dot(hn, wsd[...])


def _node_upd_last_body(h, p2, wh, wa, b1, w2, b2, g, beta, h_o):
    agg = p2[0] + p2[1]
    t = _dot(h[...], wh[...]) + _dot(agg, wa[...]) + b1[...]
    upd = _ln(_dot(_silu(t), w2[...]) + b2[...], g[...], beta[...])
    h_o[...] = h[...] + upd


def _node_upd_call(h, parts, p, wsd):
    grid = N_NODES // _NODE_TILE
    wh = p["w1"][:NODE_DIM]
    wa = p["w1"][NODE_DIM:]
    common_in = [
        _tiled(_NODE_TILE, NODE_DIM),
        pl.BlockSpec((2, _NODE_TILE, EDGE_DIM), lambda i: (0, i, 0)),
        _full((NODE_DIM, HID_NODE)), _full((EDGE_DIM, HID_NODE)),
        _full((1, HID_NODE)),
        _full((HID_NODE, NODE_DIM)), _full((1, NODE_DIM)),
        _full((1, NODE_DIM)), _full((1, NODE_DIM)),
    ]
    args = [h, parts, wh, wa, p["b1"].reshape(1, -1), p["w2"],
            p["b2"].reshape(1, -1), p["g"].reshape(1, -1),
            p["beta"].reshape(1, -1)]
    if wsd is None:
        return pl.pallas_call(
            _node_upd_last_body,
            grid=(grid,),
            in_specs=common_in,
            out_specs=_tiled(_NODE_TILE, NODE_DIM),
            out_shape=jax.ShapeDtypeStruct((N_NODES, NODE_DIM), jnp.float32),
        )(*args)
    return pl.pallas_call(
        _node_upd_proj_body,
        grid=(grid,),
        in_specs=common_in + [_full((NODE_DIM, 2 * HID_EDGE))],
        out_specs=(
            _tiled(_NODE_TILE, NODE_DIM),
            _tiled(_NODE_TILE, 2 * HID_EDGE),
        ),
        out_shape=(
            jax.ShapeDtypeStruct((N_NODES, NODE_DIM), jnp.float32),
            jax.ShapeDtypeStruct((N_NODES, 2 * HID_EDGE), jnp.float32),
        ),
    )(*(args + [wsd]))


def _final_body(h0, p0, h1, p1,
                wh0, wa0, nb10, nw20, nb20, ng0, nbeta0,
                wh1, wa1, nb11, nw21, nb21, ng1, nbeta1,
                w1a, b1a, w2a, b2a, w1b, b1b, w2b, b2b, o):
    a0 = p0[0] + p0[1]
    t0 = _dot(h0[...], wh0[...]) + _dot(a0, wa0[...]) + nb10[...]
    hn0 = h0[...] + _ln(_dot(_silu(t0), nw20[...]) + nb20[...],
                        ng0[...], nbeta0[...])
    a1 = p1[0] + p1[1]
    t1 = _dot(h1[...], wh1[...]) + _dot(a1, wa1[...]) + nb11[...]
    hn1 = h1[...] + _ln(_dot(_silu(t1), nw21[...]) + nb21[...],
                        ng1[...], nbeta1[...])
    ya = _dot(_silu(_dot(hn0, w1a[...]) + b1a[...]), w2a[...]) + b2a[...]
    yb = _dot(_silu(_dot(hn1, w1b[...]) + b1b[...]), w2b[...]) + b2b[...]
    o[...] = ya + yb


def _final_call(h0, p0, h1, p1, n0, n1, pa, pb, sp):
    grid = N_NODES // _NODE_TILE
    w2a = pa["w2"] * sp[0]
    b2a = pa["b2"].reshape(1, -1) * sp[0]
    w2b = pb["w2"] * sp[1]
    b2b = pb["b2"].reshape(1, -1) * sp[1]

    def nargs(n):
        return [n["w1"][:NODE_DIM], n["w1"][NODE_DIM:],
                n["b1"].reshape(1, -1), n["w2"], n["b2"].reshape(1, -1),
                n["g"].reshape(1, -1), n["beta"].reshape(1, -1)]

    nspecs = [_full((NODE_DIM, HID_NODE)), _full((EDGE_DIM, HID_NODE)),
              _full((1, HID_NODE)), _full((HID_NODE, NODE_DIM)),
              _full((1, NODE_DIM)), _full((1, NODE_DIM)),
              _full((1, NODE_DIM))]
    pspec = pl.BlockSpec((2, _NODE_TILE, EDGE_DIM), lambda i: (0, i, 0))
    return pl.pallas_call(
        _final_body,
        grid=(grid,),
        in_specs=(
            [_tiled(_NODE_TILE, NODE_DIM), pspec,
             _tiled(_NODE_TILE, NODE_DIM), pspec]
            + nspecs + nspecs
            + [_full((NODE_DIM, DEC_HID)), _full((1, DEC_HID)),
               _full((DEC_HID, OUT_DIM)), _full((1, OUT_DIM)),
               _full((NODE_DIM, DEC_HID)), _full((1, DEC_HID)),
               _full((DEC_HID, OUT_DIM)), _full((1, OUT_DIM))]),
        out_specs=_tiled(_NODE_TILE, OUT_DIM),
        out_shape=jax.ShapeDtypeStruct((N_NODES, OUT_DIM), jnp.float32),
    )(h0, p0, h1, p1, *nargs(n0), *nargs(n1),
      pa["w1"], pa["b1"].reshape(1, -1), w2a, b2a,
      pb["w1"], pb["b1"].reshape(1, -1), w2b, b2b)


def _decode_body(h0, h1, w1a, b1a, w2a, b2a, w1b, b1b, w2b, b2b, o):
    ya = _dot(_silu(_dot(h0[...], w1a[...]) + b1a[...]), w2a[...]) + b2a[...]
    yb = _dot(_silu(_dot(h1[...], w1b[...]) + b1b[...]), w2b[...]) + b2b[...]
    o[...] = ya + yb


def _decode_call(h0, h1, pa, pb, sp):
    grid = N_NODES // _NODE_TILE
    w2a = pa["w2"] * sp[0]
    b2a = pa["b2"].reshape(1, -1) * sp[0]
    w2b = pb["w2"] * sp[1]
    b2b = pb["b2"].reshape(1, -1) * sp[1]
    return pl.pallas_call(
        _decode_body,
        grid=(grid,),
        in_specs=[
            _tiled(_NODE_TILE, NODE_DIM), _tiled(_NODE_TILE, NODE_DIM),
            _full((NODE_DIM, DEC_HID)), _full((1, DEC_HID)),
            _full((DEC_HID, OUT_DIM)), _full((1, OUT_DIM)),
            _full((NODE_DIM, DEC_HID)), _full((1, DEC_HID)),
            _full((DEC_HID, OUT_DIM)), _full((1, OUT_DIM)),
        ],
        out_specs=_tiled(_NODE_TILE, OUT_DIM),
        out_shape=jax.ShapeDtypeStruct((N_NODES, OUT_DIM), jnp.float32),
    )(h0, h1, pa["w1"], pa["b1"].reshape(1, -1), w2a, b2a,
      pb["w1"], pb["b1"].reshape(1, -1), w2b, b2b)


@functools.lru_cache(maxsize=None)
def _sc_kernels():
    mesh = plsc.VectorSubcoreMesh(core_axis_name="c", subcore_axis_name="s",
                                  num_cores=_NC, num_subcores=_NS)

    @functools.partial(
        pl.kernel,
        mesh=mesh,
        out_type=jax.ShapeDtypeStruct((N_EDGES, HID_EDGE), jnp.float32),
        scratch_types=[
            pltpu.VMEM((_C,), jnp.int32),
            pltpu.VMEM((_C,), jnp.int32),
            pltpu.VMEM((_C,), jnp.int32),
            pltpu.VMEM((_C,), jnp.int32),
            pltpu.VMEM((_C, 2 * HID_EDGE), jnp.float32),
            pltpu.VMEM((_C, 2 * HID_EDGE), jnp.float32),
            pltpu.VMEM((_C, 2 * HID_EDGE), jnp.float32),
            pltpu.VMEM((_C, 2 * HID_EDGE), jnp.float32),
            pltpu.VMEM((_C, HID_EDGE), jnp.float32),
            pltpu.VMEM((_C, HID_EDGE), jnp.float32),
        ] + [pltpu.SemaphoreType.DMA] * 10,
    )
    def gather_add(hsd_hbm, src2_hbm, dst2_hbm, g_hbm,
                   idx_s0, idx_d0, idx_s1, idx_d1,
                   rs0, rs1, rd0, rd1, gb0, gb1,
                   si0, sj0, si1, sj1, ss0, ss1, sd0, sd1, sw0, sw1):
        wid = lax.axis_index("s") * _NC + lax.axis_index("c")
        nv = jnp.where(wid < _NCHUNK - (_JT - 1) * _NW, _JT, _JT - 1)
        slots = ((idx_s0, idx_d0, rs0, rd0, gb0, si0, sj0, ss0, sd0, sw0),
                 (idx_s1, idx_d1, rs1, rd1, gb1, si1, sj1, ss1, sd1, sw1))

        def idx_start(j, sl):
            base = (wid + _NW * j) * _C
            pltpu.async_copy(src2_hbm.at[pl.ds(base, _C)], sl[0], sl[5])
            pltpu.async_copy(dst2_hbm.at[pl.ds(base, _C)], sl[1], sl[6])

        def idx_wait(j, sl):
            base = (wid + _NW * j) * _C
            pltpu.make_async_copy(
                src2_hbm.at[pl.ds(base, _C)], sl[0], sl[5]).wait()
            pltpu.make_async_copy(
                dst2_hbm.at[pl.ds(base, _C)], sl[1], sl[6]).wait()

        def gath_start(sl):
            pltpu.async_copy(hsd_hbm.at[sl[0]], sl[2], sl[7])
            pltpu.async_copy(hsd_hbm.at[sl[1]], sl[3], sl[8])

        def gath_wait(sl):
            pltpu.make_async_copy(hsd_hbm.at[sl[0]], sl[2], sl[7]).wait()
            pltpu.make_async_copy(hsd_hbm.at[sl[1]], sl[3], sl[8]).wait()

        def wr_start(j, sl):
            base = (wid + _NW * j) * _C
            pltpu.async_copy(sl[4], g_hbm.at[pl.ds(base, _C)], sl[9])

        def wr_wait(j, sl):
            base = (wid + _NW * j) * _C
            pltpu.make_async_copy(
                sl[4], g_hbm.at[pl.ds(base, _C)], sl[9]).wait()

        for b in range(2):
            idx_start(b, slots[b])
        for b in range(2):
            idx_wait(b, slots[b])
            gath_start(slots[b])

        def body(jj, carry):
            for b in range(2):
                sl = slots[b]
                j = 2 * jj + b

                @pl.when(j < nv)
                def _():
                    gath_wait(sl)

                    @pl.when(j + 2 < nv)
                    def _():
                        idx_start(j + 2, sl)

                    @pl.when(jj > 0)
                    def _():
                        wr_wait(j - 2, sl)

                    def addrow(r, cr):
                        for u in range(HID_EDGE // 16):
                            lo = pl.ds(u * 16, 16)
                            hi = pl.ds(HID_EDGE + u * 16, 16)
                            sl[4][r, lo] = sl[2][r, lo] + sl[3][r, hi]
                        return cr

                    lax.fori_loop(0, _C, addrow, 0, unroll=4)
                    wr_start(j, sl)

                    @pl.when(j + 2 < nv)
                    def _():
                        idx_wait(j + 2, sl)
                        gath_start(sl)

                @pl.when(jnp.logical_and(j >= nv, jj > 0))
                def _():
                    wr_wait(j - 2, sl)

            return carry

        lax.fori_loop(0, _JT // 2, body, 0)
        wr_wait(_JT - 2, slots[0])

        @pl.when(nv > _JT - 1)
        def _():
            wr_wait(_JT - 1, slots[1])

    @functools.partial(
        pl.kernel,
        mesh=mesh,
        out_type=jax.ShapeDtypeStruct((_NC, _N_PAD, EDGE_DIM), jnp.float32),
        scratch_types=[
            pltpu.VMEM((_C,), jnp.int32),
            pltpu.VMEM((_C,), jnp.int32),
            pltpu.VMEM((_C, EDGE_DIM), jnp.float32),
            pltpu.VMEM((_C, EDGE_DIM), jnp.float32),
            pltpu.VMEM_SHARED((_N_PAD, EDGE_DIM), jnp.float32),
            pltpu.SemaphoreType.DMA,
            pltpu.SemaphoreType.DMA,
            pltpu.SemaphoreType.DMA,
            pltpu.SemaphoreType.DMA,
        ],
    )
    def segsum(e_hbm, dst2_hbm, p_hbm, idx0, idx1, eb0, eb1, acc,
               si0, si1, se0, se1):
        cid = lax.axis_index("c")
        sid = lax.axis_index("s")
        wid = sid * _NC + cid
        nv = jnp.where(wid < _NCHUNK - (_JT - 1) * _NW, _JT, _JT - 1)
        slots = ((idx0, eb0, si0, se0), (idx1, eb1, si1, se1))

        def ld_start(j, sl):
            base = (wid + _NW * j) * _C
            pltpu.async_copy(dst2_hbm.at[pl.ds(base, _C)], sl[0], sl[2])
            pltpu.async_copy(e_hbm.at[pl.ds(base, _C)], sl[1], sl[3])

        def ld_wait(j, sl):
            base = (wid + _NW * j) * _C
            pltpu.make_async_copy(
                dst2_hbm.at[pl.ds(base, _C)], sl[0], sl[2]).wait()
            pltpu.make_async_copy(
                e_hbm.at[pl.ds(base, _C)], sl[1], sl[3]).wait()

        zv = jnp.zeros((16,), jnp.float32)

        def zrow(r, cr):
            for u in range(EDGE_DIM // 16):
                eb0[r, pl.ds(u * 16, 16)] = zv
            return cr

        lax.fori_loop(0, 128, zrow, 0)
        for t in range(_ROWS_PER_TILE // 128):
            pltpu.sync_copy(eb0,
                            acc.at[pl.ds(sid * _ROWS_PER_TILE + t * 128, 128)])
        plsc.subcore_barrier()

        for b in range(2):
            ld_start(b, slots[b])

        def body(jj, carry):
            for b in range(2):
                sl = slots[b]
                j = 2 * jj + b

                @pl.when(j < nv)
                def _():
                    ld_wait(j, sl)
                    pltpu.sync_copy(sl[1], acc.at[sl[0]], add=True)

                    @pl.when(j + 2 < nv)
                    def _():
                        ld_start(j + 2, sl)

            return carry

        lax.fori_loop(0, _JT // 2, body, 0)
        plsc.subcore_barrier()
        pltpu.sync_copy(
            acc.at[pl.ds(sid * _ROWS_PER_TILE, _ROWS_PER_TILE)],
            p_hbm.at[cid, pl.ds(sid * _ROWS_PER_TILE, _ROWS_PER_TILE)])

    return gather_add, segsum


def _gather_add(hsd, src, dst):
    return _sc_kernels()[0](hsd, src, dst)


def _segsum(e, dst):
    return _sc_kernels()[1](e, dst)


def _split_edge_w1(w1):
    return w1[:NODE_DIM], w1[NODE_DIM:2 * NODE_DIM], w1[2 * NODE_DIM:]


def kernel(features, edge_index, edge_attr, params):
    src = edge_index[0]
    dst = edge_index[1]
    feats0 = features[0]
    sp = params["step_params"]
    models = params["models"]


    hs_, ps_ = [], []
    for i in range(len(models)):
        mp = models[i]
        blk0, blk1 = mp["blocks"]
        ws0, wd0, we0 = _split_edge_w1(blk0["edge"]["w1"])
        ws1, wd1, we1 = _split_edge_w1(blk1["edge"]["w1"])
        wsd0 = jnp.concatenate([ws0, wd0], axis=1)
        wsd1 = jnp.concatenate([ws1, wd1], axis=1)

        h, hsd = _enc_node_call(feats0[i], mp["enc_node"], wsd0)

        g = _gather_add(hsd, src, dst)
        e = _edge_enc_upd_call(edge_attr, g, mp["enc_edge"], we0,
                               blk0["edge"])
        parts = _segsum(e, dst)
        h, hsd = _node_upd_call(h, parts, blk0["node"], wsd1)

        g = _gather_add(hsd, src, dst)
        e = _edge_upd_call(e, g, we1, blk1["edge"])
        parts = _segsum(e, dst)
        hs_.append(h)
        ps_.append(parts)

    out = _final_call(hs_[0], ps_[0], hs_[1], ps_[1],
                      models[0]["blocks"][1]["node"],
                      models[1]["blocks"][1]["node"],
                      models[0]["dec"], models[1]["dec"], sp)
    return out[None]


# speedup vs baseline: 1.2792x; 1.0493x over previous
import functools

import jax
import jax.numpy as jnp
from jax import lax
from jax.experimental import pallas as pl
from jax.experimental.pallas import tpu as pltpu
from jax.experimental.pallas import tpu_sc as plsc

N_NODES = 10000
N_EDGES = 160000
FEAT = 128
NODE_DIM = 128
EDGE_DIM = 128
HID_NODE = 128
HID_EDGE = 64
DEC_HID = 64
OUT_DIM = 128

_NC = 2
_NS = 16
_NW = _NC * _NS
_C = 128
_JT = 40
_NCHUNK = N_EDGES // _C
_N_PAD = 10240
_ROWS_PER_TILE = _N_PAD // _NS
_DUMMY_NODE = _N_PAD - 1

_NODE_TILE = 2000
_EDGE_TILE = 4000


def _silu(x):
    return x * (1.0 / (1.0 + jnp.exp(-x)))


def _ln(x, g, b):
    m = jnp.mean(x, axis=-1, keepdims=True)
    d = x - m
    v = jnp.mean(d * d, axis=-1, keepdims=True)
    return d * lax.rsqrt(v + 1e-5) * g + b


def _dot(a, b):
    return jnp.dot(a, b, preferred_element_type=jnp.float32)


def _tiled(rows, cols):
    return pl.BlockSpec((rows, cols), lambda i: (i, 0))


def _full(shape):
    return pl.BlockSpec(shape, lambda i: (0,) * len(shape))


def _enc_node_body(x, w1, b1, w2, b2, g, beta, wsd, h_o, hsd_o):
    h1 = _silu(_dot(x[...], w1[...]) + b1[...])
    h = _ln(_dot(h1, w2[...]) + b2[...], g[...], beta[...])
    h_o[...] = h
    hsd_o[...] = _dot(h, wsd[...])


def _enc_node_call(x, p, wsd):
    grid = N_NODES // _NODE_TILE
    return pl.pallas_call(
        _enc_node_body,
        grid=(grid,),
        in_specs=[
            _tiled(_NODE_TILE, FEAT),
            _full((FEAT, HID_NODE)), _full((1, HID_NODE)),
            _full((HID_NODE, NODE_DIM)), _full((1, NODE_DIM)),
            _full((1, NODE_DIM)), _full((1, NODE_DIM)),
            _full((NODE_DIM, 2 * HID_EDGE)),
        ],
        out_specs=(
            _tiled(_NODE_TILE, NODE_DIM),
            _tiled(_NODE_TILE, 2 * HID_EDGE),
        ),
        out_shape=(
            jax.ShapeDtypeStruct((N_NODES, NODE_DIM), jnp.float32),
            jax.ShapeDtypeStruct((N_NODES, 2 * HID_EDGE), jnp.float32),
        ),
    )(x, p["w1"], p["b1"].reshape(1, -1), p["w2"], p["b2"].reshape(1, -1),
      p["g"].reshape(1, -1), p["beta"].reshape(1, -1), wsd)


def _enc_edge_body(x, w1, b1, w2, b2, g, beta, e_o):
    h1 = _silu(_dot(x[...], w1[...]) + b1[...])
    e_o[...] = _ln(_dot(h1, w2[...]) + b2[...], g[...], beta[...])


def _enc_edge_call(x, p):
    grid = N_EDGES // _EDGE_TILE
    return pl.pallas_call(
        _enc_edge_body,
        grid=(grid,),
        in_specs=[
            _tiled(_EDGE_TILE, 4),
            _full((4, HID_EDGE)), _full((1, HID_EDGE)),
            _full((HID_EDGE, EDGE_DIM)), _full((1, EDGE_DIM)),
            _full((1, EDGE_DIM)), _full((1, EDGE_DIM)),
        ],
        out_specs=_tiled(_EDGE_TILE, EDGE_DIM),
        out_shape=jax.ShapeDtypeStruct((N_EDGES, EDGE_DIM), jnp.float32),
    )(x, p["w1"], p["b1"].reshape(1, -1), p["w2"], p["b2"].reshape(1, -1),
      p["g"].reshape(1, -1), p["beta"].reshape(1, -1))


def _edge_enc_upd_body(x, gth, w1e, b1e, w2e, b2e, ge, betae,
                       we, b1, w2, b2, g, beta, e_o):
    enc = _ln(_dot(_silu(_dot(x[...], w1e[...]) + b1e[...]), w2e[...])
              + b2e[...], ge[...], betae[...])
    t = _dot(enc, we[...]) + gth[...] + b1[...]
    upd = _ln(_dot(_silu(t), w2[...]) + b2[...], g[...], beta[...])
    e_o[...] = enc + upd


def _edge_enc_upd_call(x, gth, pe, we, p):
    grid = N_EDGES // _EDGE_TILE
    return pl.pallas_call(
        _edge_enc_upd_body,
        grid=(grid,),
        in_specs=[
            _tiled(_EDGE_TILE, 4),
            _tiled(_EDGE_TILE, HID_EDGE),
            _full((4, HID_EDGE)), _full((1, HID_EDGE)),
            _full((HID_EDGE, EDGE_DIM)), _full((1, EDGE_DIM)),
            _full((1, EDGE_DIM)), _full((1, EDGE_DIM)),
            _full((EDGE_DIM, HID_EDGE)), _full((1, HID_EDGE)),
            _full((HID_EDGE, EDGE_DIM)), _full((1, EDGE_DIM)),
            _full((1, EDGE_DIM)), _full((1, EDGE_DIM)),
        ],
        out_specs=_tiled(_EDGE_TILE, EDGE_DIM),
        out_shape=jax.ShapeDtypeStruct((N_EDGES, EDGE_DIM), jnp.float32),
    )(x, gth,
      pe["w1"], pe["b1"].reshape(1, -1), pe["w2"], pe["b2"].reshape(1, -1),
      pe["g"].reshape(1, -1), pe["beta"].reshape(1, -1),
      we, p["b1"].reshape(1, -1), p["w2"], p["b2"].reshape(1, -1),
      p["g"].reshape(1, -1), p["beta"].reshape(1, -1))


def _edge_upd_body(e, gth, we, b1, w2, b2, g, beta, e_o):
    t = _dot(e[...], we[...]) + gth[...] + b1[...]
    upd = _ln(_dot(_silu(t), w2[...]) + b2[...], g[...], beta[...])
    e_o[...] = e[...] + upd


def _edge_upd_call(e, gth, we, p):
    grid = N_EDGES // _EDGE_TILE
    return pl.pallas_call(
        _edge_upd_body,
        grid=(grid,),
        in_specs=[
            _tiled(_EDGE_TILE, EDGE_DIM),
            _tiled(_EDGE_TILE, HID_EDGE),
            _full((EDGE_DIM, HID_EDGE)), _full((1, HID_EDGE)),
            _full((HID_EDGE, EDGE_DIM)), _full((1, EDGE_DIM)),
            _full((1, EDGE_DIM)), _full((1, EDGE_DIM)),
        ],
        out_specs=_tiled(_EDGE_TILE, EDGE_DIM),
        out_shape=jax.ShapeDtypeStruct((N_EDGES, EDGE_DIM), jnp.float32),
    )(e, gth, we, p["b1"].reshape(1, -1), p["w2"], p["b2"].reshape(1, -1),
      p["g"].reshape(1, -1), p["beta"].reshape(1, -1))


def _node_upd_proj_body(h, p2, wh, wa, b1, w2, b2, g, beta, wsd,
                        h_o, hsd_o):
    agg = p2[0] + p2[1]
    t = _dot(h[...], wh[...]) + _dot(agg, wa[...]) + b1[...]
    upd = _ln(_dot(_silu(t), w2[...]) + b2[...], g[...], beta[...])
    hn = h[...] + upd
    h_o[...] = hn
    hsd_o[...] = _dot(hn, wsd[...])


def _node_upd_last_body(h, p2, wh, wa, b1, w2, b2, g, beta, h_o):
    agg = p2[0] + p2[1]
    t = _dot(h[...], wh[...]) + _dot(agg, wa[...]) + b1[...]
    upd = _ln(_dot(_silu(t), w2[...]) + b2[...], g[...], beta[...])
    h_o[...] = h[...] + upd


def _node_upd_call(h, parts, p, wsd):
    grid = N_NODES // _NODE_TILE
    wh = p["w1"][:NODE_DIM]
    wa = p["w1"][NODE_DIM:]
    common_in = [
        _tiled(_NODE_TILE, NODE_DIM),
        pl.BlockSpec((2, _NODE_TILE, EDGE_DIM), lambda i: (0, i, 0)),
        _full((NODE_DIM, HID_NODE)), _full((EDGE_DIM, HID_NODE)),
        _full((1, HID_NODE)),
        _full((HID_NODE, NODE_DIM)), _full((1, NODE_DIM)),
        _full((1, NODE_DIM)), _full((1, NODE_DIM)),
    ]
    args = [h, parts, wh, wa, p["b1"].reshape(1, -1), p["w2"],
            p["b2"].reshape(1, -1), p["g"].reshape(1, -1),
            p["beta"].reshape(1, -1)]
    if wsd is None:
        return pl.pallas_call(
            _node_upd_last_body,
            grid=(grid,),
            in_specs=common_in,
            out_specs=_tiled(_NODE_TILE, NODE_DIM),
            out_shape=jax.ShapeDtypeStruct((N_NODES, NODE_DIM), jnp.float32),
        )(*args)
    return pl.pallas_call(
        _node_upd_proj_body,
        grid=(grid,),
        in_specs=common_in + [_full((NODE_DIM, 2 * HID_EDGE))],
        out_specs=(
            _tiled(_NODE_TILE, NODE_DIM),
            _tiled(_NODE_TILE, 2 * HID_EDGE),
        ),
        out_shape=(
            jax.ShapeDtypeStruct((N_NODES, NODE_DIM), jnp.float32),
            jax.ShapeDtypeStruct((N_NODES, 2 * HID_EDGE), jnp.float32),
        ),
    )(*(args + [wsd]))


def _final_body(h0, p0, h1, p1,
                wh0, wa0, nb10, nw20, nb20, ng0, nbeta0,
                wh1, wa1, nb11, nw21, nb21, ng1, nbeta1,
                w1a, b1a, w2a, b2a, w1b, b1b, w2b, b2b, o):
    a0 = p0[0] + p0[1]
    t0 = _dot(h0[...], wh0[...]) + _dot(a0, wa0[...]) + nb10[...]
    hn0 = h0[...] + _ln(_dot(_silu(t0), nw20[...]) + nb20[...],
                        ng0[...], nbeta0[...])
    a1 = p1[0] + p1[1]
    t1 = _dot(h1[...], wh1[...]) + _dot(a1, wa1[...]) + nb11[...]
    hn1 = h1[...] + _ln(_dot(_silu(t1), nw21[...]) + nb21[...],
                        ng1[...], nbeta1[...])
    ya = _dot(_silu(_dot(hn0, w1a[...]) + b1a[...]), w2a[...]) + b2a[...]
    yb = _dot(_silu(_dot(hn1, w1b[...]) + b1b[...]), w2b[...]) + b2b[...]
    o[...] = ya + yb


def _final_call(h0, p0, h1, p1, n0, n1, pa, pb, sp):
    grid = N_NODES // _NODE_TILE
    w2a = pa["w2"] * sp[0]
    b2a = pa["b2"].reshape(1, -1) * sp[0]
    w2b = pb["w2"] * sp[1]
    b2b = pb["b2"].reshape(1, -1) * sp[1]

    def nargs(n):
        return [n["w1"][:NODE_DIM], n["w1"][NODE_DIM:],
                n["b1"].reshape(1, -1), n["w2"], n["b2"].reshape(1, -1),
                n["g"].reshape(1, -1), n["beta"].reshape(1, -1)]

    nspecs = [_full((NODE_DIM, HID_NODE)), _full((EDGE_DIM, HID_NODE)),
              _full((1, HID_NODE)), _full((HID_NODE, NODE_DIM)),
              _full((1, NODE_DIM)), _full((1, NODE_DIM)),
              _full((1, NODE_DIM))]
    pspec = pl.BlockSpec((2, _NODE_TILE, EDGE_DIM), lambda i: (0, i, 0))
    return pl.pallas_call(
        _final_body,
        grid=(grid,),
        in_specs=(
            [_tiled(_NODE_TILE, NODE_DIM), pspec,
             _tiled(_NODE_TILE, NODE_DIM), pspec]
            + nspecs + nspecs
            + [_full((NODE_DIM, DEC_HID)), _full((1, DEC_HID)),
               _full((DEC_HID, OUT_DIM)), _full((1, OUT_DIM)),
               _full((NODE_DIM, DEC_HID)), _full((1, DEC_HID)),
               _full((DEC_HID, OUT_DIM)), _full((1, OUT_DIM))]),
        out_specs=_tiled(_NODE_TILE, OUT_DIM),
        out_shape=jax.ShapeDtypeStruct((N_NODES, OUT_DIM), jnp.float32),
    )(h0, p0, h1, p1, *nargs(n0), *nargs(n1),
      pa["w1"], pa["b1"].reshape(1, -1), w2a, b2a,
      pb["w1"], pb["b1"].reshape(1, -1), w2b, b2b)


def _decode_body(h0, h1, w1a, b1a, w2a, b2a, w1b, b1b, w2b, b2b, o):
    ya = _dot(_silu(_dot(h0[...], w1a[...]) + b1a[...]), w2a[...]) + b2a[...]
    yb = _dot(_silu(_dot(h1[...], w1b[...]) + b1b[...]), w2b[...]) + b2b[...]
    o[...] = ya + yb


def _decode_call(h0, h1, pa, pb, sp):
    grid = N_NODES // _NODE_TILE
    w2a = pa["w2"] * sp[0]
    b2a = pa["b2"].reshape(1, -1) * sp[0]
    w2b = pb["w2"] * sp[1]
    b2b = pb["b2"].reshape(1, -1) * sp[1]
    return pl.pallas_call(
        _decode_body,
        grid=(grid,),
        in_specs=[
            _tiled(_NODE_TILE, NODE_DIM), _tiled(_NODE_TILE, NODE_DIM),
            _full((NODE_DIM, DEC_HID)), _full((1, DEC_HID)),
            _full((DEC_HID, OUT_DIM)), _full((1, OUT_DIM)),
            _full((NODE_DIM, DEC_HID)), _full((1, DEC_HID)),
            _full((DEC_HID, OUT_DIM)), _full((1, OUT_DIM)),
        ],
        out_specs=_tiled(_NODE_TILE, OUT_DIM),
        out_shape=jax.ShapeDtypeStruct((N_NODES, OUT_DIM), jnp.float32),
    )(h0, h1, pa["w1"], pa["b1"].reshape(1, -1), w2a, b2a,
      pb["w1"], pb["b1"].reshape(1, -1), w2b, b2b)


@functools.lru_cache(maxsize=None)
def _sc_kernels():
    mesh = plsc.VectorSubcoreMesh(core_axis_name="c", subcore_axis_name="s",
                                  num_cores=_NC, num_subcores=_NS)

    @functools.partial(
        pl.kernel,
        mesh=mesh,
        out_type=jax.ShapeDtypeStruct((N_EDGES, HID_EDGE), jnp.float32),
        scratch_types=[
            pltpu.VMEM((_C,), jnp.int32),
            pltpu.VMEM((_C,), jnp.int32),
            pltpu.VMEM((_C,), jnp.int32),
            pltpu.VMEM((_C,), jnp.int32),
            pltpu.VMEM((_C, 2 * HID_EDGE), jnp.float32),
            pltpu.VMEM((_C, 2 * HID_EDGE), jnp.float32),
            pltpu.VMEM((_C, 2 * HID_EDGE), jnp.float32),
            pltpu.VMEM((_C, 2 * HID_EDGE), jnp.float32),
            pltpu.VMEM((_C, HID_EDGE), jnp.float32),
            pltpu.VMEM((_C, HID_EDGE), jnp.float32),
        ] + [pltpu.SemaphoreType.DMA] * 10,
    )
    def gather_add(hsd_hbm, src2_hbm, dst2_hbm, g_hbm,
                   idx_s0, idx_d0, idx_s1, idx_d1,
                   rs0, rs1, rd0, rd1, gb0, gb1,
                   si0, sj0, si1, sj1, ss0, ss1, sd0, sd1, sw0, sw1):
        wid = lax.axis_index("s") * _NC + lax.axis_index("c")
        nv = jnp.where(wid < _NCHUNK - (_JT - 1) * _NW, _JT, _JT - 1)
        slots = ((idx_s0, idx_d0, rs0, rd0, gb0, si0, sj0, ss0, sd0, sw0),
                 (idx_s1, idx_d1, rs1, rd1, gb1, si1, sj1, ss1, sd1, sw1))

        def idx_start(j, sl):
            base = (wid + _NW * j) * _C
            pltpu.async_copy(src2_hbm.at[pl.ds(base, _C)], sl[0], sl[5])
            pltpu.async_copy(dst2_hbm.at[pl.ds(base, _C)], sl[1], sl[6])

        def idx_wait(j, sl):
            base = (wid + _NW * j) * _C
            pltpu.make_async_copy(
                src2_hbm.at[pl.ds(base, _C)], sl[0], sl[5]).wait()
            pltpu.make_async_copy(
                dst2_hbm.at[pl.ds(base, _C)], sl[1], sl[6]).wait()

        def gath_start(sl):
            pltpu.async_copy(hsd_hbm.at[sl[0]], sl[2], sl[7])
            pltpu.async_copy(hsd_hbm.at[sl[1]], sl[3], sl[8])

        def gath_wait(sl):
            pltpu.make_async_copy(hsd_hbm.at[sl[0]], sl[2], sl[7]).wait()
            pltpu.make_async_copy(hsd_hbm.at[sl[1]], sl[3], sl[8]).wait()

        def wr_start(j, sl):
            base = (wid + _NW * j) * _C
            pltpu.async_copy(sl[4], g_hbm.at[pl.ds(base, _C)], sl[9])

        def wr_wait(j, sl):
            base = (wid + _NW * j) * _C
            pltpu.make_async_copy(
                sl[4], g_hbm.at[pl.ds(base, _C)], sl[9]).wait()

        for b in range(2):
            idx_start(b, slots[b])
        for b in range(2):
            idx_wait(b, slots[b])
            gath_start(slots[b])

        def body(jj, carry):
            for b in range(2):
                sl = slots[b]
                j = 2 * jj + b

                @pl.when(j < nv)
                def _():
                    gath_wait(sl)

                    @pl.when(j + 2 < nv)
                    def _():
                        idx_start(j + 2, sl)

                    @pl.when(jj > 0)
                    def _():
                        wr_wait(j - 2, sl)

                    def addrow(r, cr):
                        for u in range(HID_EDGE // 16):
                            lo = pl.ds(u * 16, 16)
                            hi = pl.ds(HID_EDGE + u * 16, 16)
                            sl[4][r, lo] = sl[2][r, lo] + sl[3][r, hi]
                        return cr

                    lax.fori_loop(0, _C, addrow, 0, unroll=4)
                    wr_start(j, sl)

                    @pl.when(j + 2 < nv)
                    def _():
                        idx_wait(j + 2, sl)
                        gath_start(sl)

                @pl.when(jnp.logical_and(j >= nv, jj > 0))
                def _():
                    wr_wait(j - 2, sl)

            return carry

        lax.fori_loop(0, _JT // 2, body, 0)
        wr_wait(_JT - 2, slots[0])

        @pl.when(nv > _JT - 1)
        def _():
            wr_wait(_JT - 1, slots[1])

    @functools.partial(
        pl.kernel,
        mesh=mesh,
        out_type=jax.ShapeDtypeStruct((_NC, _N_PAD, EDGE_DIM), jnp.float32),
        scratch_types=[
            pltpu.VMEM((_C,), jnp.int32),
            pltpu.VMEM((_C,), jnp.int32),
            pltpu.VMEM((_C, EDGE_DIM), jnp.float32),
            pltpu.VMEM((_C, EDGE_DIM), jnp.float32),
            pltpu.VMEM_SHARED((_N_PAD, EDGE_DIM), jnp.float32),
            pltpu.SemaphoreType.DMA,
            pltpu.SemaphoreType.DMA,
            pltpu.SemaphoreType.DMA,
            pltpu.SemaphoreType.DMA,
        ],
    )
    def segsum(e_hbm, dst2_hbm, p_hbm, idx0, idx1, eb0, eb1, acc,
               si0, si1, se0, se1):
        cid = lax.axis_index("c")
        sid = lax.axis_index("s")
        wid = sid * _NC + cid
        nv = jnp.where(wid < _NCHUNK - (_JT - 1) * _NW, _JT, _JT - 1)
        slots = ((idx0, eb0, si0, se0), (idx1, eb1, si1, se1))

        def ld_start(j, sl):
            base = (wid + _NW * j) * _C
            pltpu.async_copy(dst2_hbm.at[pl.ds(base, _C)], sl[0], sl[2])
            pltpu.async_copy(e_hbm.at[pl.ds(base, _C)], sl[1], sl[3])

        def ld_wait(j, sl):
            base = (wid + _NW * j) * _C
            pltpu.make_async_copy(
                dst2_hbm.at[pl.ds(base, _C)], sl[0], sl[2]).wait()
            pltpu.make_async_copy(
                e_hbm.at[pl.ds(base, _C)], sl[1], sl[3]).wait()

        zv = jnp.zeros((16,), jnp.float32)

        def zrow(r, cr):
            for u in range(EDGE_DIM // 16):
                eb0[r, pl.ds(u * 16, 16)] = zv
            return cr

        lax.fori_loop(0, 128, zrow, 0)
        for t in range(_ROWS_PER_TILE // 128):
            pltpu.sync_copy(eb0,
                            acc.at[pl.ds(sid * _ROWS_PER_TILE + t * 128, 128)])
        plsc.subcore_barrier()

        for b in range(2):
            ld_start(b, slots[b])

        def body(jj, carry):
            for b in range(2):
                sl = slots[b]
                j = 2 * jj + b

                @pl.when(j < nv)
                def _():
                    ld_wait(j, sl)
                    pltpu.sync_copy(sl[1], acc.at[sl[0]], add=True)

                    @pl.when(j + 2 < nv)
                    def _():
                        ld_start(j + 2, sl)

            return carry

        lax.fori_loop(0, _JT // 2, body, 0)
        plsc.subcore_barrier()
        pltpu.sync_copy(
            acc.at[pl.ds(sid * _ROWS_PER_TILE, _ROWS_PER_TILE)],
            p_hbm.at[cid, pl.ds(sid * _ROWS_PER_TILE, _ROWS_PER_TILE)])

    return gather_add, segsum


def _gather_add(hsd, src, dst):
    return _sc_kernels()[0](hsd, src, dst)


def _segsum(e, dst):
    return _sc_kernels()[1](e, dst)


def _split_edge_w1(w1):
    return w1[:NODE_DIM], w1[NODE_DIM:2 * NODE_DIM], w1[2 * NODE_DIM:]


def kernel(features, edge_index, edge_attr, params):
    src = edge_index[0]
    dst = edge_index[1]
    feats0 = features[0]
    sp = params["step_params"]
    models = params["models"]


    hs_, ps_ = [], []
    for i in range(len(models)):
        mp = models[i]
        blk0, blk1 = mp["blocks"]
        ws0, wd0, we0 = _split_edge_w1(blk0["edge"]["w1"])
        ws1, wd1, we1 = _split_edge_w1(blk1["edge"]["w1"])
        wsd0 = jnp.concatenate([ws0, wd0], axis=1)
        wsd1 = jnp.concatenate([ws1, wd1], axis=1)

        h, hsd = _enc_node_call(feats0[i], mp["enc_node"], wsd0)

        g = _gather_add(hsd, src, dst)
        e = _edge_enc_upd_call(edge_attr, g, mp["enc_edge"], we0,
                               blk0["edge"])
        parts = _segsum(e, dst)
        h, hsd = _node_upd_call(h, parts, blk0["node"], wsd1)

        g = _gather_add(hsd, src, dst)
        e = _edge_upd_call(e, g, we1, blk1["edge"])
        parts = _segsum(e, dst)
        hs_.append(h)
        ps_.append(parts)

    out = _final_call(hs_[0], ps_[0], hs_[1], ps_[1],
                      models[0]["blocks"][1]["node"],
                      models[1]["blocks"][1]["node"],
                      models[0]["dec"], models[1]["dec"], sp)
    return out[None]


# speedup vs baseline: 1.2945x; 1.0120x over previous
import functools

import jax
import jax.numpy as jnp
from jax import lax
from jax.experimental import pallas as pl
from jax.experimental.pallas import tpu as pltpu
from jax.experimental.pallas import tpu_sc as plsc

N_NODES = 10000
N_EDGES = 160000
FEAT = 128
NODE_DIM = 128
EDGE_DIM = 128
HID_NODE = 128
HID_EDGE = 64
DEC_HID = 64
OUT_DIM = 128

_NC = 2
_NS = 16
_NW = _NC * _NS
_C = 128
_JT = 40
_NCHUNK = N_EDGES // _C
_N_PAD = 10240
_ROWS_PER_TILE = _N_PAD // _NS
_DUMMY_NODE = _N_PAD - 1

_NODE_TILE = 5000
_EDGE_TILE = 8000


def _silu(x):
    return x * (1.0 / (1.0 + jnp.exp(-x)))


def _ln(x, g, b):
    m = jnp.mean(x, axis=-1, keepdims=True)
    d = x - m
    v = jnp.mean(d * d, axis=-1, keepdims=True)
    return d * lax.rsqrt(v + 1e-5) * g + b


def _dot(a, b):
    return jnp.dot(a, b, preferred_element_type=jnp.float32)


def _tiled(rows, cols):
    return pl.BlockSpec((rows, cols), lambda i: (i, 0))


def _full(shape):
    return pl.BlockSpec(shape, lambda i: (0,) * len(shape))


def _enc_node_body(x, w1, b1, w2, b2, g, beta, wsd, h_o, hsd_o):
    h1 = _silu(_dot(x[...], w1[...]) + b1[...])
    h = _ln(_dot(h1, w2[...]) + b2[...], g[...], beta[...])
    h_o[...] = h
    hsd_o[...] = _dot(h, wsd[...])


def _enc_node_call(x, p, wsd):
    grid = N_NODES // _NODE_TILE
    return pl.pallas_call(
        _enc_node_body,
        grid=(grid,),
        in_specs=[
            _tiled(_NODE_TILE, FEAT),
            _full((FEAT, HID_NODE)), _full((1, HID_NODE)),
            _full((HID_NODE, NODE_DIM)), _full((1, NODE_DIM)),
            _full((1, NODE_DIM)), _full((1, NODE_DIM)),
            _full((NODE_DIM, 2 * HID_EDGE)),
        ],
        out_specs=(
            _tiled(_NODE_TILE, NODE_DIM),
            _tiled(_NODE_TILE, 2 * HID_EDGE),
        ),
        out_shape=(
            jax.ShapeDtypeStruct((N_NODES, NODE_DIM), jnp.float32),
            jax.ShapeDtypeStruct((N_NODES, 2 * HID_EDGE), jnp.float32),
        ),
    )(x, p["w1"], p["b1"].reshape(1, -1), p["w2"], p["b2"].reshape(1, -1),
      p["g"].reshape(1, -1), p["beta"].reshape(1, -1), wsd)


def _enc_edge_body(x, w1, b1, w2, b2, g, beta, e_o):
    h1 = _silu(_dot(x[...], w1[...]) + b1[...])
    e_o[...] = _ln(_dot(h1, w2[...]) + b2[...], g[...], beta[...])


def _enc_edge_call(x, p):
    grid = N_EDGES // _EDGE_TILE
    return pl.pallas_call(
        _enc_edge_body,
        grid=(grid,),
        in_specs=[
            _tiled(_EDGE_TILE, 4),
            _full((4, HID_EDGE)), _full((1, HID_EDGE)),
            _full((HID_EDGE, EDGE_DIM)), _full((1, EDGE_DIM)),
            _full((1, EDGE_DIM)), _full((1, EDGE_DIM)),
        ],
        out_specs=_tiled(_EDGE_TILE, EDGE_DIM),
        out_shape=jax.ShapeDtypeStruct((N_EDGES, EDGE_DIM), jnp.float32),
    )(x, p["w1"], p["b1"].reshape(1, -1), p["w2"], p["b2"].reshape(1, -1),
      p["g"].reshape(1, -1), p["beta"].reshape(1, -1))


def _edge_enc_upd_body(x, gth, w1e, b1e, w2e, b2e, ge, betae,
                       we, b1, w2, b2, g, beta, e_o):
    enc = _ln(_dot(_silu(_dot(x[...], w1e[...]) + b1e[...]), w2e[...])
              + b2e[...], ge[...], betae[...])
    t = _dot(enc, we[...]) + gth[...] + b1[...]
    upd = _ln(_dot(_silu(t), w2[...]) + b2[...], g[...], beta[...])
    e_o[...] = enc + upd


def _edge_enc_upd_call(x, gth, pe, we, p):
    grid = N_EDGES // _EDGE_TILE
    return pl.pallas_call(
        _edge_enc_upd_body,
        grid=(grid,),
        in_specs=[
            _tiled(_EDGE_TILE, 4),
            _tiled(_EDGE_TILE, HID_EDGE),
            _full((4, HID_EDGE)), _full((1, HID_EDGE)),
            _full((HID_EDGE, EDGE_DIM)), _full((1, EDGE_DIM)),
            _full((1, EDGE_DIM)), _full((1, EDGE_DIM)),
            _full((EDGE_DIM, HID_EDGE)), _full((1, HID_EDGE)),
            _full((HID_EDGE, EDGE_DIM)), _full((1, EDGE_DIM)),
            _full((1, EDGE_DIM)), _full((1, EDGE_DIM)),
        ],
        out_specs=_tiled(_EDGE_TILE, EDGE_DIM),
        out_shape=jax.ShapeDtypeStruct((N_EDGES, EDGE_DIM), jnp.float32),
    )(x, gth,
      pe["w1"], pe["b1"].reshape(1, -1), pe["w2"], pe["b2"].reshape(1, -1),
      pe["g"].reshape(1, -1), pe["beta"].reshape(1, -1),
      we, p["b1"].reshape(1, -1), p["w2"], p["b2"].reshape(1, -1),
      p["g"].reshape(1, -1), p["beta"].reshape(1, -1))


def _edge_upd_body(e, gth, we, b1, w2, b2, g, beta, e_o):
    t = _dot(e[...], we[...]) + gth[...] + b1[...]
    upd = _ln(_dot(_silu(t), w2[...]) + b2[...], g[...], beta[...])
    e_o[...] = e[...] + upd


def _edge_upd_call(e, gth, we, p):
    grid = N_EDGES // _EDGE_TILE
    return pl.pallas_call(
        _edge_upd_body,
        grid=(grid,),
        in_specs=[
            _tiled(_EDGE_TILE, EDGE_DIM),
            _tiled(_EDGE_TILE, HID_EDGE),
            _full((EDGE_DIM, HID_EDGE)), _full((1, HID_EDGE)),
            _full((HID_EDGE, EDGE_DIM)), _full((1, EDGE_DIM)),
            _full((1, EDGE_DIM)), _full((1, EDGE_DIM)),
        ],
        out_specs=_tiled(_EDGE_TILE, EDGE_DIM),
        out_shape=jax.ShapeDtypeStruct((N_EDGES, EDGE_DIM), jnp.float32),
    )(e, gth, we, p["b1"].reshape(1, -1), p["w2"], p["b2"].reshape(1, -1),
      p["g"].reshape(1, -1), p["beta"].reshape(1, -1))


def _node_upd_proj_body(h, p2, wh, wa, b1, w2, b2, g, beta, wsd,
                        h_o, hsd_o):
    agg = p2[0] + p2[1]
    t = _dot(h[...], wh[...]) + _dot(agg, wa[...]) + b1[...]
    upd = _ln(_dot(_silu(t), w2[...]) + b2[...], g[...], beta[...])
    hn = h[...] + upd
    h_o[...] = hn
    hsd_o[...] = _dot(hn, wsd[...])


def _node_upd_last_body(h, p2, wh, wa, b1, w2, b2, g, beta, h_o):
    agg = p2[0] + p2[1]
    t = _dot(h[...], wh[...]) + _dot(agg, wa[...]) + b1[...]
    upd = _ln(_dot(_silu(t), w2[...]) + b2[...], g[...], beta[...])
    h_o[...] = h[...] + upd


def _node_upd_call(h, parts, p, wsd):
    grid = N_NODES // _NODE_TILE
    wh = p["w1"][:NODE_DIM]
    wa = p["w1"][NODE_DIM:]
    common_in = [
        _tiled(_NODE_TILE, NODE_DIM),
        pl.BlockSpec((2, _NODE_TILE, EDGE_DIM), lambda i: (0, i, 0)),
        _full((NODE_DIM, HID_NODE)), _full((EDGE_DIM, HID_NODE)),
        _full((1, HID_NODE)),
        _full((HID_NODE, NODE_DIM)), _full((1, NODE_DIM)),
        _full((1, NODE_DIM)), _full((1, NODE_DIM)),
    ]
    args = [h, parts, wh, wa, p["b1"].reshape(1, -1), p["w2"],
            p["b2"].reshape(1, -1), p["g"].reshape(1, -1),
            p["beta"].reshape(1, -1)]
    if wsd is None:
        return pl.pallas_call(
            _node_upd_last_body,
            grid=(grid,),
            in_specs=common_in,
            out_specs=_tiled(_NODE_TILE, NODE_DIM),
            out_shape=jax.ShapeDtypeStruct((N_NODES, NODE_DIM), jnp.float32),
        )(*args)
    return pl.pallas_call(
        _node_upd_proj_body,
        grid=(grid,),
        in_specs=common_in + [_full((NODE_DIM, 2 * HID_EDGE))],
        out_specs=(
            _tiled(_NODE_TILE, NODE_DIM),
            _tiled(_NODE_TILE, 2 * HID_EDGE),
        ),
        out_shape=(
            jax.ShapeDtypeStruct((N_NODES, NODE_DIM), jnp.float32),
            jax.ShapeDtypeStruct((N_NODES, 2 * HID_EDGE), jnp.float32),
        ),
    )(*(args + [wsd]))


def _final_body(h0, p0, h1, p1,
                wh0, wa0, nb10, nw20, nb20, ng0, nbeta0,
                wh1, wa1, nb11, nw21, nb21, ng1, nbeta1,
                w1a, b1a, w2a, b2a, w1b, b1b, w2b, b2b, o):
    a0 = p0[0] + p0[1]
    t0 = _dot(h0[...], wh0[...]) + _dot(a0, wa0[...]) + nb10[...]
    hn0 = h0[...] + _ln(_dot(_silu(t0), nw20[...]) + nb20[...],
                        ng0[...], nbeta0[...])
    a1 = p1[0] + p1[1]
    t1 = _dot(h1[...], wh1[...]) + _dot(a1, wa1[...]) + nb11[...]
    hn1 = h1[...] + _ln(_dot(_silu(t1), nw21[...]) + nb21[...],
                        ng1[...], nbeta1[...])
    ya = _dot(_silu(_dot(hn0, w1a[...]) + b1a[...]), w2a[...]) + b2a[...]
    yb = _dot(_silu(_dot(hn1, w1b[...]) + b1b[...]), w2b[...]) + b2b[...]
    o[...] = ya + yb


def _final_call(h0, p0, h1, p1, n0, n1, pa, pb, sp):
    grid = N_NODES // _NODE_TILE
    w2a = pa["w2"] * sp[0]
    b2a = pa["b2"].reshape(1, -1) * sp[0]
    w2b = pb["w2"] * sp[1]
    b2b = pb["b2"].reshape(1, -1) * sp[1]

    def nargs(n):
        return [n["w1"][:NODE_DIM], n["w1"][NODE_DIM:],
                n["b1"].reshape(1, -1), n["w2"], n["b2"].reshape(1, -1),
                n["g"].reshape(1, -1), n["beta"].reshape(1, -1)]

    nspecs = [_full((NODE_DIM, HID_NODE)), _full((EDGE_DIM, HID_NODE)),
              _full((1, HID_NODE)), _full((HID_NODE, NODE_DIM)),
              _full((1, NODE_DIM)), _full((1, NODE_DIM)),
              _full((1, NODE_DIM))]
    pspec = pl.BlockSpec((2, _NODE_TILE, EDGE_DIM), lambda i: (0, i, 0))
    return pl.pallas_call(
        _final_body,
        grid=(grid,),
        in_specs=(
            [_tiled(_NODE_TILE, NODE_DIM), pspec,
             _tiled(_NODE_TILE, NODE_DIM), pspec]
            + nspecs + nspecs
            + [_full((NODE_DIM, DEC_HID)), _full((1, DEC_HID)),
               _full((DEC_HID, OUT_DIM)), _full((1, OUT_DIM)),
               _full((NODE_DIM, DEC_HID)), _full((1, DEC_HID)),
               _full((DEC_HID, OUT_DIM)), _full((1, OUT_DIM))]),
        out_specs=_tiled(_NODE_TILE, OUT_DIM),
        out_shape=jax.ShapeDtypeStruct((N_NODES, OUT_DIM), jnp.float32),
    )(h0, p0, h1, p1, *nargs(n0), *nargs(n1),
      pa["w1"], pa["b1"].reshape(1, -1), w2a, b2a,
      pb["w1"], pb["b1"].reshape(1, -1), w2b, b2b)


def _decode_body(h0, h1, w1a, b1a, w2a, b2a, w1b, b1b, w2b, b2b, o):
    ya = _dot(_silu(_dot(h0[...], w1a[...]) + b1a[...]), w2a[...]) + b2a[...]
    yb = _dot(_silu(_dot(h1[...], w1b[...]) + b1b[...]), w2b[...]) + b2b[...]
    o[...] = ya + yb


def _decode_call(h0, h1, pa, pb, sp):
    grid = N_NODES // _NODE_TILE
    w2a = pa["w2"] * sp[0]
    b2a = pa["b2"].reshape(1, -1) * sp[0]
    w2b = pb["w2"] * sp[1]
    b2b = pb["b2"].reshape(1, -1) * sp[1]
    return pl.pallas_call(
        _decode_body,
        grid=(grid,),
        in_specs=[
            _tiled(_NODE_TILE, NODE_DIM), _tiled(_NODE_TILE, NODE_DIM),
            _full((NODE_DIM, DEC_HID)), _full((1, DEC_HID)),
            _full((DEC_HID, OUT_DIM)), _full((1, OUT_DIM)),
            _full((NODE_DIM, DEC_HID)), _full((1, DEC_HID)),
            _full((DEC_HID, OUT_DIM)), _full((1, OUT_DIM)),
        ],
        out_specs=_tiled(_NODE_TILE, OUT_DIM),
        out_shape=jax.ShapeDtypeStruct((N_NODES, OUT_DIM), jnp.float32),
    )(h0, h1, pa["w1"], pa["b1"].reshape(1, -1), w2a, b2a,
      pb["w1"], pb["b1"].reshape(1, -1), w2b, b2b)


@functools.lru_cache(maxsize=None)
def _sc_kernels():
    mesh = plsc.VectorSubcoreMesh(core_axis_name="c", subcore_axis_name="s",
                                  num_cores=_NC, num_subcores=_NS)

    @functools.partial(
        pl.kernel,
        mesh=mesh,
        out_type=jax.ShapeDtypeStruct((N_EDGES, HID_EDGE), jnp.float32),
        scratch_types=[
            pltpu.VMEM((_C,), jnp.int32),
            pltpu.VMEM((_C,), jnp.int32),
            pltpu.VMEM((_C,), jnp.int32),
            pltpu.VMEM((_C,), jnp.int32),
            pltpu.VMEM((_C, 2 * HID_EDGE), jnp.float32),
            pltpu.VMEM((_C, 2 * HID_EDGE), jnp.float32),
            pltpu.VMEM((_C, 2 * HID_EDGE), jnp.float32),
            pltpu.VMEM((_C, 2 * HID_EDGE), jnp.float32),
            pltpu.VMEM((_C, HID_EDGE), jnp.float32),
            pltpu.VMEM((_C, HID_EDGE), jnp.float32),
        ] + [pltpu.SemaphoreType.DMA] * 10,
    )
    def gather_add(hsd_hbm, src2_hbm, dst2_hbm, g_hbm,
                   idx_s0, idx_d0, idx_s1, idx_d1,
                   rs0, rs1, rd0, rd1, gb0, gb1,
                   si0, sj0, si1, sj1, ss0, ss1, sd0, sd1, sw0, sw1):
        wid = lax.axis_index("s") * _NC + lax.axis_index("c")
        nv = jnp.where(wid < _NCHUNK - (_JT - 1) * _NW, _JT, _JT - 1)
        slots = ((idx_s0, idx_d0, rs0, rd0, gb0, si0, sj0, ss0, sd0, sw0),
                 (idx_s1, idx_d1, rs1, rd1, gb1, si1, sj1, ss1, sd1, sw1))

        def idx_start(j, sl):
            base = (wid + _NW * j) * _C
            pltpu.async_copy(src2_hbm.at[pl.ds(base, _C)], sl[0], sl[5])
            pltpu.async_copy(dst2_hbm.at[pl.ds(base, _C)], sl[1], sl[6])

        def idx_wait(j, sl):
            base = (wid + _NW * j) * _C
            pltpu.make_async_copy(
                src2_hbm.at[pl.ds(base, _C)], sl[0], sl[5]).wait()
            pltpu.make_async_copy(
                dst2_hbm.at[pl.ds(base, _C)], sl[1], sl[6]).wait()

        def gath_start(sl):
            pltpu.async_copy(hsd_hbm.at[sl[0]], sl[2], sl[7])
            pltpu.async_copy(hsd_hbm.at[sl[1]], sl[3], sl[8])

        def gath_wait(sl):
            pltpu.make_async_copy(hsd_hbm.at[sl[0]], sl[2], sl[7]).wait()
            pltpu.make_async_copy(hsd_hbm.at[sl[1]], sl[3], sl[8]).wait()

        def wr_start(j, sl):
            base = (wid + _NW * j) * _C
            pltpu.async_copy(sl[4], g_hbm.at[pl.ds(base, _C)], sl[9])

        def wr_wait(j, sl):
            base = (wid + _NW * j) * _C
            pltpu.make_async_copy(
                sl[4], g_hbm.at[pl.ds(base, _C)], sl[9]).wait()

        for b in range(2):
            idx_start(b, slots[b])
        for b in range(2):
            idx_wait(b, slots[b])
            gath_start(slots[b])

        def body(jj, carry):
            for b in range(2):
                sl = slots[b]
                j = 2 * jj + b

                @pl.when(j < nv)
                def _():
                    gath_wait(sl)

                    @pl.when(j + 2 < nv)
                    def _():
                        idx_start(j + 2, sl)

                    @pl.when(jj > 0)
                    def _():
                        wr_wait(j - 2, sl)

                    def addrow(r, cr):
                        for u in range(HID_EDGE // 16):
                            lo = pl.ds(u * 16, 16)
                            hi = pl.ds(HID_EDGE + u * 16, 16)
                            sl[4][r, lo] = sl[2][r, lo] + sl[3][r, hi]
                        return cr

                    lax.fori_loop(0, _C, addrow, 0, unroll=4)
                    wr_start(j, sl)

                    @pl.when(j + 2 < nv)
                    def _():
                        idx_wait(j + 2, sl)
                        gath_start(sl)

                @pl.when(jnp.logical_and(j >= nv, jj > 0))
                def _():
                    wr_wait(j - 2, sl)

            return carry

        lax.fori_loop(0, _JT // 2, body, 0)
        wr_wait(_JT - 2, slots[0])

        @pl.when(nv > _JT - 1)
        def _():
            wr_wait(_JT - 1, slots[1])

    @functools.partial(
        pl.kernel,
        mesh=mesh,
        out_type=jax.ShapeDtypeStruct((_NC, _N_PAD, EDGE_DIM), jnp.float32),
        scratch_types=[
            pltpu.VMEM((_C,), jnp.int32),
            pltpu.VMEM((_C,), jnp.int32),
            pltpu.VMEM((_C, EDGE_DIM), jnp.float32),
            pltpu.VMEM((_C, EDGE_DIM), jnp.float32),
            pltpu.VMEM_SHARED((_N_PAD, EDGE_DIM), jnp.float32),
            pltpu.SemaphoreType.DMA,
            pltpu.SemaphoreType.DMA,
            pltpu.SemaphoreType.DMA,
            pltpu.SemaphoreType.DMA,
        ],
    )
    def segsum(e_hbm, dst2_hbm, p_hbm, idx0, idx1, eb0, eb1, acc,
               si0, si1, se0, se1):
        cid = lax.axis_index("c")
        sid = lax.axis_index("s")
        wid = sid * _NC + cid
        nv = jnp.where(wid < _NCHUNK - (_JT - 1) * _NW, _JT, _JT - 1)
        slots = ((idx0, eb0, si0, se0), (idx1, eb1, si1, se1))

        def ld_start(j, sl):
            base = (wid + _NW * j) * _C
            pltpu.async_copy(dst2_hbm.at[pl.ds(base, _C)], sl[0], sl[2])
            pltpu.async_copy(e_hbm.at[pl.ds(base, _C)], sl[1], sl[3])

        def ld_wait(j, sl):
            base = (wid + _NW * j) * _C
            pltpu.make_async_copy(
                dst2_hbm.at[pl.ds(base, _C)], sl[0], sl[2]).wait()
            pltpu.make_async_copy(
                e_hbm.at[pl.ds(base, _C)], sl[1], sl[3]).wait()

        zv = jnp.zeros((16,), jnp.float32)

        def zrow(r, cr):
            for u in range(EDGE_DIM // 16):
                eb0[r, pl.ds(u * 16, 16)] = zv
            return cr

        lax.fori_loop(0, 128, zrow, 0)
        for t in range(_ROWS_PER_TILE // 128):
            pltpu.sync_copy(eb0,
                            acc.at[pl.ds(sid * _ROWS_PER_TILE + t * 128, 128)])
        plsc.subcore_barrier()

        for b in range(2):
            ld_start(b, slots[b])

        def body(jj, carry):
            for b in range(2):
                sl = slots[b]
                j = 2 * jj + b

                @pl.when(j < nv)
                def _():
                    ld_wait(j, sl)
                    pltpu.sync_copy(sl[1], acc.at[sl[0]], add=True)

                    @pl.when(j + 2 < nv)
                    def _():
                        ld_start(j + 2, sl)

            return carry

        lax.fori_loop(0, _JT // 2, body, 0)
        plsc.subcore_barrier()
        pltpu.sync_copy(
            acc.at[pl.ds(sid * _ROWS_PER_TILE, _ROWS_PER_TILE)],
            p_hbm.at[cid, pl.ds(sid * _ROWS_PER_TILE, _ROWS_PER_TILE)])

    return gather_add, segsum


def _gather_add(hsd, src, dst):
    return _sc_kernels()[0](hsd, src, dst)


def _segsum(e, dst):
    return _sc_kernels()[1](e, dst)


def _split_edge_w1(w1):
    return w1[:NODE_DIM], w1[NODE_DIM:2 * NODE_DIM], w1[2 * NODE_DIM:]


def kernel(features, edge_index, edge_attr, params):
    src = edge_index[0]
    dst = edge_index[1]
    feats0 = features[0]
    sp = params["step_params"]
    models = params["models"]


    hs_, ps_ = [], []
    for i in range(len(models)):
        mp = models[i]
        blk0, blk1 = mp["blocks"]
        ws0, wd0, we0 = _split_edge_w1(blk0["edge"]["w1"])
        ws1, wd1, we1 = _split_edge_w1(blk1["edge"]["w1"])
        wsd0 = jnp.concatenate([ws0, wd0], axis=1)
        wsd1 = jnp.concatenate([ws1, wd1], axis=1)

        h, hsd = _enc_node_call(feats0[i], mp["enc_node"], wsd0)

        g = _gather_add(hsd, src, dst)
        e = _edge_enc_upd_call(edge_attr, g, mp["enc_edge"], we0,
                               blk0["edge"])
        parts = _segsum(e, dst)
        h, hsd = _node_upd_call(h, parts, blk0["node"], wsd1)

        g = _gather_add(hsd, src, dst)
        e = _edge_upd_call(e, g, we1, blk1["edge"])
        parts = _segsum(e, dst)
        hs_.append(h)
        ps_.append(parts)

    out = _final_call(hs_[0], ps_[0], hs_[1], ps_[1],
                      models[0]["blocks"][1]["node"],
                      models[1]["blocks"][1]["node"],
                      models[0]["dec"], models[1]["dec"], sp)
    return out[None]
